# Initial kernel scaffold; baseline (speedup 1.0000x reference)
import jax
import jax.numpy as jnp
from jax.experimental import pallas as pl


def kernel(x, edge_index, ptr, params):
    raise NotImplementedError("write your pallas kernel here")


# speedup vs baseline: 2.7557x; 2.7557x over previous
import functools
import math

import jax
import jax.numpy as jnp
from jax import lax
from jax.experimental import pallas as pl
from jax.experimental.pallas import tpu as pltpu
from jax.experimental.pallas import tpu_sc as plsc

D = 128
ACC_W = 144
NC, NS = 2, 16
NW = NC * NS
CHUNK = 32
SUB = 16
NSUB = CHUNK // SUB
ROW_BLK = 1280


def _edge_body(C, hs_hbm, hd_hbm, cidx_hbm, att_hbm,
               feat_hbm, s_hbm,
               acc_sh, ibuf, dbuf, hs_buf, hd_buf, out_buf, att_v, zrow,
               t_ref, s_loc, sg0h, sg0d, sg1h, sg1d, ss0, ss1, si0, si1):
    npad = hs_hbm.shape[0]
    cid = lax.axis_index("c")
    sid = lax.axis_index("s")
    wid = sid * NC + cid

    pltpu.sync_copy(att_hbm, att_v)

    z16 = jnp.zeros((16,), jnp.float32)
    for r in range(SUB):
        for c in range(D // 16):
            zrow[r, pl.ds(c * 16, 16)] = z16

    rows_per_sub = npad // NS

    def _zero(k, carry):
        pltpu.sync_copy(zrow, acc_sh.at[pl.ds(sid * rows_per_sub + k * SUB, SUB)])
        return carry
    lax.fori_loop(0, rows_per_sub // SUB, _zero, 0)

    def _zero_s(k, carry):
        s_loc[pl.ds(k * 16, 16)] = z16
        return carry
    lax.fori_loop(0, npad // 16, _zero_s, 0)

    plsc.subcore_barrier()

    lanes = lax.iota(jnp.int32, SUB)

    def _loop(chunk, carry):
        for b in range(1):
            pltpu.sync_copy(cidx_hbm.at[wid, chunk], ibuf.at[b])
            pltpu.sync_copy(hs_hbm.at[ibuf.at[b, 0]], hs_buf.at[b])
            pltpu.sync_copy(hd_hbm.at[ibuf.at[b, 1]], hd_buf.at[b])

            for k in range(CHUNK // 16):
                dbuf[b, pl.ds(k * 16, 16)] = ibuf[b, 1, pl.ds(k * 16, 16)]

            def _sub(sb, cc):
                base = sb * SUB
                att_regs = [att_v[pl.ds(c * 16, 16)] for c in range(D // 16)]
                for e_i in range(SUB):
                    r = base + e_i
                    acc = None
                    for c in range(D // 16):
                        a = hs_buf[b, r, pl.ds(c * 16, 16)]
                        t = a + hd_buf[b, r, pl.ds(c * 16, 16)]
                        zl = jnp.maximum(t, 0.2 * t)
                        term = zl * att_regs[c]
                        acc = term if acc is None else acc + term
                    t_ref[e_i, :] = acc
                e_vec = None
                for c in range(SUB):
                    col = plsc.load_gather(
                        t_ref, [lanes, jnp.full((SUB,), c, jnp.int32)])
                    e_vec = col if e_vec is None else e_vec + col
                w_vec = jnp.exp(e_vec)
                row_idx = lanes + base
                dst_vec = dbuf[b, pl.ds(base, SUB)]
                for li in range(SUB):
                    plsc.addupdate_scatter(s_loc, [dst_vec], w_vec,
                                           mask=lanes == li)
                hs2 = hs_buf.at[b]
                ob2 = out_buf.at[b]
                for f in range(D):
                    fidx = jnp.full((SUB,), f, jnp.int32)
                    colv = plsc.load_gather(hs2, [row_idx, fidx])
                    plsc.store_scatter(ob2, [row_idx, fidx], colv * w_vec)
                return cc
            lax.fori_loop(0, NSUB, _sub, 0)

            pltpu.sync_copy(out_buf.at[b], acc_sh.at[dbuf.at[b]], add=True)
        return carry
    lax.fori_loop(0, C, _loop, 0)

    plsc.subcore_barrier()
    pltpu.sync_copy(acc_sh.at[pl.ds(sid * rows_per_sub, rows_per_sub)],
                    feat_hbm.at[cid, pl.ds(sid * rows_per_sub, rows_per_sub)])
    pltpu.sync_copy(s_loc, s_hbm.at[wid])


@functools.cache
def _make_edge_kernel(C, npad):
    mesh = plsc.VectorSubcoreMesh(core_axis_name="c", subcore_axis_name="s")
    return pl.kernel(
        functools.partial(_edge_body, C),
        out_type=(jax.ShapeDtypeStruct((NC, npad, D), jnp.float32),
                  jax.ShapeDtypeStruct((NW, npad), jnp.float32)),
        mesh=mesh,
        compiler_params=pltpu.CompilerParams(needs_layout_passes=False),
        scratch_types=[
            pltpu.VMEM_SHARED((npad, D), jnp.float32),
            pltpu.VMEM((2, 2, CHUNK), jnp.int32),
            pltpu.VMEM((2, CHUNK), jnp.int32),
            pltpu.VMEM((2, CHUNK, D), jnp.float32),
            pltpu.VMEM((2, CHUNK, D), jnp.float32),
            pltpu.VMEM((2, CHUNK, D), jnp.float32),
            pltpu.VMEM((D,), jnp.float32),
            pltpu.VMEM((SUB, D), jnp.float32),
            pltpu.VMEM((SUB, SUB), jnp.float32),
            pltpu.VMEM((npad,), jnp.float32),
            pltpu.SemaphoreType.DMA,
            pltpu.SemaphoreType.DMA,
            pltpu.SemaphoreType.DMA,
            pltpu.SemaphoreType.DMA,
            pltpu.SemaphoreType.DMA,
            pltpu.SemaphoreType.DMA,
            pltpu.SemaphoreType.DMA,
            pltpu.SemaphoreType.DMA,
        ],
    )


def _prep_body(x_ref, wp_ref, bp_ref, ws_ref, wd_ref, h0_ref, hs_ref, hd_ref):
    h0 = jnp.dot(x_ref[...], wp_ref[...],
                 preferred_element_type=jnp.float32) + bp_ref[...]
    h0_ref[...] = h0
    hs_ref[...] = jnp.dot(h0, ws_ref[...], preferred_element_type=jnp.float32)
    hd_ref[...] = jnp.dot(h0, wd_ref[...], preferred_element_type=jnp.float32)


def _ln_rows(x, g, b):
    m = jnp.mean(x, axis=-1, keepdims=True)
    v = jnp.mean((x - m) ** 2, axis=-1, keepdims=True)
    return (x - m) / jnp.sqrt(v + 1e-5) * g + b


def _fin_mid_body(acc_ref, s_ref, bias_ref, g_ref, b_ref, ws_ref, wd_ref,
                  h_ref, hs_ref, hd_ref):
    a = acc_ref[0] + acc_ref[1]
    s = jnp.sum(s_ref[...], axis=0)[:, None]
    out = a / (s + 1e-30) + bias_ref[...]
    h = jnp.maximum(_ln_rows(out, g_ref[...], b_ref[...]), 0.0)
    h_ref[...] = h
    hs_ref[...] = jnp.dot(h, ws_ref[...], preferred_element_type=jnp.float32)
    hd_ref[...] = jnp.dot(h, wd_ref[...], preferred_element_type=jnp.float32)


def _fin_last_body(acc_ref, s_ref, bias_ref, h0_ref, wl_ref, bl_ref, bng_ref,
                   bnb_ref, xn_ref):
    a = acc_ref[0] + acc_ref[1]
    s = jnp.sum(s_ref[...], axis=0)[:, None]
    h = a / (s + 1e-30) + bias_ref[...]
    xn = jnp.dot(h0_ref[...] + h, wl_ref[...],
                 preferred_element_type=jnp.float32) + bl_ref[...]
    xn_ref[...] = xn / jnp.sqrt(1.0 + 1e-5) * bng_ref[...] + bnb_ref[...]


def _tr_body(seq_ref, mask_ref, wq_ref, bq_ref, wk_ref, bk_ref, wv_ref, bv_ref,
             wo_ref, bo_ref, w1_ref, b1_ref, w2_ref, b2_ref,
             g1_ref, gb1_ref, g2_ref, gb2_ref, out_ref):
    xx = seq_ref[0]
    q = jnp.dot(xx, wq_ref[...], preferred_element_type=jnp.float32) + bq_ref[...]
    k = jnp.dot(xx, wk_ref[...], preferred_element_type=jnp.float32) + bk_ref[...]
    v = jnp.dot(xx, wv_ref[...], preferred_element_type=jnp.float32) + bv_ref[...]
    dh = 32
    scale = 1.0 / math.sqrt(float(dh))
    outs = []
    for h in range(4):
        qh = q[:, h * dh:(h + 1) * dh]
        kh = k[:, h * dh:(h + 1) * dh]
        vh = v[:, h * dh:(h + 1) * dh]
        sc = lax.dot_general(qh, kh, (((1,), (1,)), ((), ())),
                             preferred_element_type=jnp.float32) * scale
        sc = sc + mask_ref[...]
        m = jnp.max(sc, axis=1, keepdims=True)
        p = jnp.exp(sc - m)
        p = p / jnp.sum(p, axis=1, keepdims=True)
        outs.append(jnp.dot(p, vh, preferred_element_type=jnp.float32))
    sa = jnp.concatenate(outs, axis=1)
    sa = jnp.dot(sa, wo_ref[...], preferred_element_type=jnp.float32) + bo_ref[...]
    s1 = _ln_rows(xx + sa, g1_ref[...], gb1_ref[...])
    ff = jnp.maximum(jnp.dot(s1, w1_ref[...],
                             preferred_element_type=jnp.float32) + b1_ref[...], 0.0)
    ff = jnp.dot(ff, w2_ref[...], preferred_element_type=jnp.float32) + b2_ref[...]
    out_ref[0] = _ln_rows(s1 + ff, g2_ref[...], gb2_ref[...])


def _row_kernel(body, npad, n_out, extra_specs, grid_rows=ROW_BLK):
    ngrid = npad // grid_rows
    rb = lambda i: (i, 0)
    full2 = lambda i: (0, 0)
    return pl.pallas_call(
        body,
        grid=(ngrid,),
        in_specs=extra_specs,
        out_specs=[pl.BlockSpec((grid_rows, D), rb)] * n_out,
        out_shape=[jax.ShapeDtypeStruct((npad, D), jnp.float32)] * n_out,
    )


def kernel(x, edge_index, ptr, params):
    p = params
    n = x.shape[0]
    npad = ((n + NS * SUB * 40 - 1) // (NS * SUB * 40)) * (NS * SUB * 40)
    b_graphs = ptr.shape[0] - 1
    seq_len = n // b_graphs

    xp = jnp.zeros((npad, D), jnp.float32).at[:n].set(x)
    loop = jnp.arange(n, dtype=edge_index.dtype)
    e_all = edge_index.shape[1] + n
    c_chunks = -(-e_all // (NW * CHUNK))
    e_pad = NW * CHUNK * c_chunks
    fill = jnp.full((e_pad - e_all,), n, edge_index.dtype)
    src_r = jnp.concatenate([edge_index[0], loop, fill]).reshape(NW, c_chunks, CHUNK)
    dst_r = jnp.concatenate([edge_index[1], loop, fill]).reshape(NW, c_chunks, CHUNK)
    cidx = jnp.stack([src_r, dst_r], axis=2)

    rb = lambda i: (i, 0)
    w_spec = pl.BlockSpec((D, D), lambda i: (0, 0))
    b_spec = pl.BlockSpec((1, D), lambda i: (0, 0))
    row_spec = pl.BlockSpec((ROW_BLK, D), rb)
    acc_spec = pl.BlockSpec((NC, ROW_BLK, D), lambda i: (0, i, 0))
    s_spec = pl.BlockSpec((NW, ROW_BLK), lambda i: (0, i))

    prep = _row_kernel(_prep_body, npad, 3,
                       [row_spec, w_spec, b_spec, w_spec, w_spec])
    h0, hs, hd = prep(xp, p["W_proj"], p["b_proj"].reshape(1, D),
                      p["gat_W_src"][0], p["gat_W_dst"][0])

    edge_k = _make_edge_kernel(c_chunks, npad)
    nl = p["gat_W_src"].shape[0]

    fin_mid = pl.pallas_call(
        _fin_mid_body,
        grid=(npad // ROW_BLK,),
        in_specs=[acc_spec, s_spec, b_spec, b_spec, b_spec, w_spec, w_spec],
        out_specs=[row_spec] * 3,
        out_shape=[jax.ShapeDtypeStruct((npad, D), jnp.float32)] * 3,
    )
    for l in range(nl - 1):
        acc, svec = edge_k(hs, hd, cidx, p["gat_att"][l])
        h, hs, hd = fin_mid(acc, svec, p["gat_bias"][l].reshape(1, D),
                            p["gat_ln_g"][l].reshape(1, D),
                            p["gat_ln_b"][l].reshape(1, D),
                            p["gat_W_src"][l + 1], p["gat_W_dst"][l + 1])

    acc, svec = edge_k(hs, hd, cidx, p["gat_att"][nl - 1])
    fin_last = pl.pallas_call(
        _fin_last_body,
        grid=(npad // ROW_BLK,),
        in_specs=[acc_spec, s_spec, b_spec, row_spec, w_spec, b_spec, b_spec,
                  b_spec],
        out_specs=[row_spec],
        out_shape=[jax.ShapeDtypeStruct((npad, D), jnp.float32)],
    )
    (xn,) = fin_last(acc, svec, p["gat_bias"][nl - 1].reshape(1, D), h0,
                     p["W_lin"], p["b_lin"].reshape(1, D),
                     p["bn_g"].reshape(1, D), p["bn_b"].reshape(1, D))

    X = xn[:n].reshape(b_graphs, seq_len, D)
    tok = lambda t: jnp.tile(t[None, None, :], (b_graphs, 1, 1))
    seq = jnp.concatenate([tok(p["CLS"]), X, tok(p["RING"]), tok(p["END"])],
                          axis=1)
    s_real = seq.shape[1]
    s_pad = 128
    seqp = jnp.zeros((b_graphs, s_pad, D), jnp.float32).at[:, :s_real].set(seq)
    mask = jnp.where(jnp.arange(s_pad) < s_real, 0.0, -1e30)
    mask = mask.astype(jnp.float32).reshape(1, s_pad)

    sblk = pl.BlockSpec((1, s_pad, D), lambda i: (i, 0, 0))
    full2 = lambda i: (0, 0)
    wspec = pl.BlockSpec((D, D), full2)
    bspec = pl.BlockSpec((1, D), full2)
    mspec = pl.BlockSpec((1, s_pad), full2)
    w1spec = pl.BlockSpec((D, 1024), full2)
    b1spec = pl.BlockSpec((1, 1024), full2)
    w2spec = pl.BlockSpec((1024, D), full2)
    tr = pl.pallas_call(
        _tr_body,
        grid=(b_graphs,),
        in_specs=[sblk, mspec,
                  wspec, bspec, wspec, bspec, wspec, bspec, wspec, bspec,
                  w1spec, b1spec, w2spec, bspec,
                  bspec, bspec, bspec, bspec],
        out_specs=[sblk],
        out_shape=[jax.ShapeDtypeStruct((b_graphs, s_pad, D), jnp.float32)],
    )
    (out,) = tr(seqp, mask,
                p["Wq"], p["bq"].reshape(1, D), p["Wk"], p["bk"].reshape(1, D),
                p["Wv"], p["bv"].reshape(1, D), p["Wo"], p["bo"].reshape(1, D),
                p["W1"], p["b1"].reshape(1, 1024), p["W2"], p["b2"].reshape(1, D),
                p["ln1_g"].reshape(1, D), p["ln1_b"].reshape(1, D),
                p["ln2_g"].reshape(1, D), p["ln2_b"].reshape(1, D))
    return out[:, :s_real]


# speedup vs baseline: 3.3441x; 1.2135x over previous
import functools
import math

import jax
import jax.numpy as jnp
from jax import lax
from jax.experimental import pallas as pl
from jax.experimental.pallas import tpu as pltpu
from jax.experimental.pallas import tpu_sc as plsc

D = 128
ACC_W = 144
NC, NS = 2, 16
NW = NC * NS
CHUNK = 32
SUB = 16
NSUB = CHUNK // SUB
ROW_BLK = 1280


def _edge_body(C, hs_hbm, hd_hbm, cidx_hbm, att_hbm,
               feat_hbm, s_hbm,
               acc_sh, ibuf, dbuf, hs_buf, hd_buf, out_buf, att_v, zrow,
               t_ref, s_loc, sg0h, sg0d, sg1h, sg1d):
    npad = hs_hbm.shape[0]
    cid = lax.axis_index("c")
    sid = lax.axis_index("s")
    wid = sid * NC + cid

    pltpu.sync_copy(att_hbm, att_v)

    z16 = jnp.zeros((16,), jnp.float32)
    for r in range(SUB):
        for c in range(D // 16):
            zrow[r, pl.ds(c * 16, 16)] = z16

    rows_per_sub = npad // NS

    def _zero(k, carry):
        pltpu.sync_copy(zrow, acc_sh.at[pl.ds(sid * rows_per_sub + k * SUB, SUB)])
        return carry
    lax.fori_loop(0, rows_per_sub // SUB, _zero, 0)

    def _zero_s(k, carry):
        s_loc[pl.ds(k * 16, 16)] = z16
        return carry
    lax.fori_loop(0, npad // 16, _zero_s, 0)

    plsc.subcore_barrier()

    lanes = lax.iota(jnp.int32, SUB)
    g_h = (sg0h, sg1h)
    g_d = (sg0d, sg1d)

    pltpu.sync_copy(cidx_hbm.at[wid, 0], ibuf.at[0])
    pltpu.sync_copy(hs_hbm.at[ibuf.at[0, 0]], hs_buf.at[0])
    pltpu.sync_copy(hd_hbm.at[ibuf.at[0, 1]], hd_buf.at[0])

    def _loop(i, carry):
        def _prefetch(chunk, b):
            pltpu.sync_copy(cidx_hbm.at[wid, chunk], ibuf.at[b])
            return (pltpu.async_copy(hs_hbm.at[ibuf.at[b, 0]], hs_buf.at[b],
                                     g_h[b]),
                    pltpu.async_copy(hd_hbm.at[ibuf.at[b, 1]], hd_buf.at[b],
                                     g_d[b]))

        def _chunk_compute(b):
            for k in range(CHUNK // 16):
                dbuf[b, pl.ds(k * 16, 16)] = ibuf[b, 1, pl.ds(k * 16, 16)]

            def _sub(sb, cc):
                base = sb * SUB
                att_regs = [att_v[pl.ds(c * 16, 16)] for c in range(D // 16)]
                for e_i in range(SUB):
                    r = base + e_i
                    acc = None
                    for c in range(D // 16):
                        a = hs_buf[b, r, pl.ds(c * 16, 16)]
                        t = a + hd_buf[b, r, pl.ds(c * 16, 16)]
                        zl = jnp.maximum(t, 0.2 * t)
                        term = zl * att_regs[c]
                        acc = term if acc is None else acc + term
                    t_ref[e_i, :] = acc
                e_vec = None
                for c in range(SUB):
                    col = plsc.load_gather(
                        t_ref, [lanes, jnp.full((SUB,), c, jnp.int32)])
                    e_vec = col if e_vec is None else e_vec + col
                w_vec = jnp.exp(e_vec)
                row_idx = lanes + base
                dst_vec = dbuf[b, pl.ds(base, SUB)]
                for li in range(SUB):
                    plsc.addupdate_scatter(s_loc, [dst_vec], w_vec,
                                           mask=lanes == li)
                hs2 = hs_buf.at[b]
                ob2 = out_buf.at[b]
                for f in range(D):
                    fidx = jnp.full((SUB,), f, jnp.int32)
                    colv = plsc.load_gather(hs2, [row_idx, fidx])
                    plsc.store_scatter(ob2, [row_idx, fidx], colv * w_vec)
                return cc
            lax.fori_loop(0, NSUB, _sub, 0)

            pltpu.sync_copy(out_buf.at[b], acc_sh.at[dbuf.at[b]], add=True)

        chunk_a = 2 * i
        d_b = _prefetch(chunk_a + 1, 1)
        _chunk_compute(0)
        d_b[0].wait()
        d_b[1].wait()
        d_a2 = _prefetch(chunk_a + 2, 0)
        _chunk_compute(1)
        d_a2[0].wait()
        d_a2[1].wait()
        return carry
    lax.fori_loop(0, C // 2, _loop, 0)

    plsc.subcore_barrier()
    pltpu.sync_copy(acc_sh.at[pl.ds(sid * rows_per_sub, rows_per_sub)],
                    feat_hbm.at[cid, pl.ds(sid * rows_per_sub, rows_per_sub)])
    pltpu.sync_copy(s_loc, s_hbm.at[wid])


@functools.cache
def _make_edge_kernel(C, npad):
    mesh = plsc.VectorSubcoreMesh(core_axis_name="c", subcore_axis_name="s")
    return pl.kernel(
        functools.partial(_edge_body, C),
        out_type=(jax.ShapeDtypeStruct((NC, npad, D), jnp.float32),
                  jax.ShapeDtypeStruct((NW, npad), jnp.float32)),
        mesh=mesh,
        compiler_params=pltpu.CompilerParams(needs_layout_passes=False),
        scratch_types=[
            pltpu.VMEM_SHARED((npad, D), jnp.float32),
            pltpu.VMEM((2, 2, CHUNK), jnp.int32),
            pltpu.VMEM((2, CHUNK), jnp.int32),
            pltpu.VMEM((2, CHUNK, D), jnp.float32),
            pltpu.VMEM((2, CHUNK, D), jnp.float32),
            pltpu.VMEM((2, CHUNK, D), jnp.float32),
            pltpu.VMEM((D,), jnp.float32),
            pltpu.VMEM((SUB, D), jnp.float32),
            pltpu.VMEM((SUB, SUB), jnp.float32),
            pltpu.VMEM((npad,), jnp.float32),
            pltpu.SemaphoreType.DMA,
            pltpu.SemaphoreType.DMA,
            pltpu.SemaphoreType.DMA,
            pltpu.SemaphoreType.DMA,
        ],
    )


def _prep_body(x_ref, wp_ref, bp_ref, ws_ref, wd_ref, h0_ref, hs_ref, hd_ref):
    h0 = jnp.dot(x_ref[...], wp_ref[...],
                 preferred_element_type=jnp.float32) + bp_ref[...]
    h0_ref[...] = h0
    hs_ref[...] = jnp.dot(h0, ws_ref[...], preferred_element_type=jnp.float32)
    hd_ref[...] = jnp.dot(h0, wd_ref[...], preferred_element_type=jnp.float32)


def _ln_rows(x, g, b):
    m = jnp.mean(x, axis=-1, keepdims=True)
    v = jnp.mean((x - m) ** 2, axis=-1, keepdims=True)
    return (x - m) / jnp.sqrt(v + 1e-5) * g + b


def _fin_mid_body(acc_ref, s_ref, bias_ref, g_ref, b_ref, ws_ref, wd_ref,
                  h_ref, hs_ref, hd_ref):
    a = acc_ref[0] + acc_ref[1]
    s = jnp.sum(s_ref[...], axis=0)[:, None]
    out = a / (s + 1e-30) + bias_ref[...]
    h = jnp.maximum(_ln_rows(out, g_ref[...], b_ref[...]), 0.0)
    h_ref[...] = h
    hs_ref[...] = jnp.dot(h, ws_ref[...], preferred_element_type=jnp.float32)
    hd_ref[...] = jnp.dot(h, wd_ref[...], preferred_element_type=jnp.float32)


def _fin_last_body(acc_ref, s_ref, bias_ref, h0_ref, wl_ref, bl_ref, bng_ref,
                   bnb_ref, xn_ref):
    a = acc_ref[0] + acc_ref[1]
    s = jnp.sum(s_ref[...], axis=0)[:, None]
    h = a / (s + 1e-30) + bias_ref[...]
    xn = jnp.dot(h0_ref[...] + h, wl_ref[...],
                 preferred_element_type=jnp.float32) + bl_ref[...]
    xn_ref[...] = xn / jnp.sqrt(1.0 + 1e-5) * bng_ref[...] + bnb_ref[...]


def _tr_body(seq_ref, mask_ref, wq_ref, bq_ref, wk_ref, bk_ref, wv_ref, bv_ref,
             wo_ref, bo_ref, w1_ref, b1_ref, w2_ref, b2_ref,
             g1_ref, gb1_ref, g2_ref, gb2_ref, out_ref):
    xx = seq_ref[0]
    q = jnp.dot(xx, wq_ref[...], preferred_element_type=jnp.float32) + bq_ref[...]
    k = jnp.dot(xx, wk_ref[...], preferred_element_type=jnp.float32) + bk_ref[...]
    v = jnp.dot(xx, wv_ref[...], preferred_element_type=jnp.float32) + bv_ref[...]
    dh = 32
    scale = 1.0 / math.sqrt(float(dh))
    outs = []
    for h in range(4):
        qh = q[:, h * dh:(h + 1) * dh]
        kh = k[:, h * dh:(h + 1) * dh]
        vh = v[:, h * dh:(h + 1) * dh]
        sc = lax.dot_general(qh, kh, (((1,), (1,)), ((), ())),
                             preferred_element_type=jnp.float32) * scale
        sc = sc + mask_ref[...]
        m = jnp.max(sc, axis=1, keepdims=True)
        p = jnp.exp(sc - m)
        p = p / jnp.sum(p, axis=1, keepdims=True)
        outs.append(jnp.dot(p, vh, preferred_element_type=jnp.float32))
    sa = jnp.concatenate(outs, axis=1)
    sa = jnp.dot(sa, wo_ref[...], preferred_element_type=jnp.float32) + bo_ref[...]
    s1 = _ln_rows(xx + sa, g1_ref[...], gb1_ref[...])
    ff = jnp.maximum(jnp.dot(s1, w1_ref[...],
                             preferred_element_type=jnp.float32) + b1_ref[...], 0.0)
    ff = jnp.dot(ff, w2_ref[...], preferred_element_type=jnp.float32) + b2_ref[...]
    out_ref[0] = _ln_rows(s1 + ff, g2_ref[...], gb2_ref[...])


def _row_kernel(body, npad, n_out, extra_specs, grid_rows=ROW_BLK):
    ngrid = npad // grid_rows
    rb = lambda i: (i, 0)
    full2 = lambda i: (0, 0)
    return pl.pallas_call(
        body,
        grid=(ngrid,),
        in_specs=extra_specs,
        out_specs=[pl.BlockSpec((grid_rows, D), rb)] * n_out,
        out_shape=[jax.ShapeDtypeStruct((npad, D), jnp.float32)] * n_out,
    )


def kernel(x, edge_index, ptr, params):
    p = params
    n = x.shape[0]
    npad = ((n + NS * SUB * 40 - 1) // (NS * SUB * 40)) * (NS * SUB * 40)
    b_graphs = ptr.shape[0] - 1
    seq_len = n // b_graphs

    xp = jnp.zeros((npad, D), jnp.float32).at[:n].set(x)
    loop = jnp.arange(n, dtype=edge_index.dtype)
    e_all = edge_index.shape[1] + n
    c_chunks = -(-e_all // (NW * CHUNK))
    e_pad = NW * CHUNK * c_chunks
    fill = jnp.full((e_pad - e_all,), n, edge_index.dtype)
    src_r = jnp.concatenate([edge_index[0], loop, fill]).reshape(NW, c_chunks, CHUNK)
    dst_r = jnp.concatenate([edge_index[1], loop, fill]).reshape(NW, c_chunks, CHUNK)
    cidx = jnp.stack([src_r, dst_r], axis=2)
    cidx = jnp.concatenate(
        [cidx, jnp.full((NW, 2, 2, CHUNK), n, edge_index.dtype)], axis=1)

    rb = lambda i: (i, 0)
    w_spec = pl.BlockSpec((D, D), lambda i: (0, 0))
    b_spec = pl.BlockSpec((1, D), lambda i: (0, 0))
    row_spec = pl.BlockSpec((ROW_BLK, D), rb)
    acc_spec = pl.BlockSpec((NC, ROW_BLK, D), lambda i: (0, i, 0))
    s_spec = pl.BlockSpec((NW, ROW_BLK), lambda i: (0, i))

    prep = _row_kernel(_prep_body, npad, 3,
                       [row_spec, w_spec, b_spec, w_spec, w_spec])
    h0, hs, hd = prep(xp, p["W_proj"], p["b_proj"].reshape(1, D),
                      p["gat_W_src"][0], p["gat_W_dst"][0])

    edge_k = _make_edge_kernel(c_chunks, npad)
    nl = p["gat_W_src"].shape[0]

    fin_mid = pl.pallas_call(
        _fin_mid_body,
        grid=(npad // ROW_BLK,),
        in_specs=[acc_spec, s_spec, b_spec, b_spec, b_spec, w_spec, w_spec],
        out_specs=[row_spec] * 3,
        out_shape=[jax.ShapeDtypeStruct((npad, D), jnp.float32)] * 3,
    )
    for l in range(nl - 1):
        acc, svec = edge_k(hs, hd, cidx, p["gat_att"][l])
        h, hs, hd = fin_mid(acc, svec, p["gat_bias"][l].reshape(1, D),
                            p["gat_ln_g"][l].reshape(1, D),
                            p["gat_ln_b"][l].reshape(1, D),
                            p["gat_W_src"][l + 1], p["gat_W_dst"][l + 1])

    acc, svec = edge_k(hs, hd, cidx, p["gat_att"][nl - 1])
    fin_last = pl.pallas_call(
        _fin_last_body,
        grid=(npad // ROW_BLK,),
        in_specs=[acc_spec, s_spec, b_spec, row_spec, w_spec, b_spec, b_spec,
                  b_spec],
        out_specs=[row_spec],
        out_shape=[jax.ShapeDtypeStruct((npad, D), jnp.float32)],
    )
    (xn,) = fin_last(acc, svec, p["gat_bias"][nl - 1].reshape(1, D), h0,
                     p["W_lin"], p["b_lin"].reshape(1, D),
                     p["bn_g"].reshape(1, D), p["bn_b"].reshape(1, D))

    X = xn[:n].reshape(b_graphs, seq_len, D)
    tok = lambda t: jnp.tile(t[None, None, :], (b_graphs, 1, 1))
    seq = jnp.concatenate([tok(p["CLS"]), X, tok(p["RING"]), tok(p["END"])],
                          axis=1)
    s_real = seq.shape[1]
    s_pad = 128
    seqp = jnp.zeros((b_graphs, s_pad, D), jnp.float32).at[:, :s_real].set(seq)
    mask = jnp.where(jnp.arange(s_pad) < s_real, 0.0, -1e30)
    mask = mask.astype(jnp.float32).reshape(1, s_pad)

    sblk = pl.BlockSpec((1, s_pad, D), lambda i: (i, 0, 0))
    full2 = lambda i: (0, 0)
    wspec = pl.BlockSpec((D, D), full2)
    bspec = pl.BlockSpec((1, D), full2)
    mspec = pl.BlockSpec((1, s_pad), full2)
    w1spec = pl.BlockSpec((D, 1024), full2)
    b1spec = pl.BlockSpec((1, 1024), full2)
    w2spec = pl.BlockSpec((1024, D), full2)
    tr = pl.pallas_call(
        _tr_body,
        grid=(b_graphs,),
        in_specs=[sblk, mspec,
                  wspec, bspec, wspec, bspec, wspec, bspec, wspec, bspec,
                  w1spec, b1spec, w2spec, bspec,
                  bspec, bspec, bspec, bspec],
        out_specs=[sblk],
        out_shape=[jax.ShapeDtypeStruct((b_graphs, s_pad, D), jnp.float32)],
    )
    (out,) = tr(seqp, mask,
                p["Wq"], p["bq"].reshape(1, D), p["Wk"], p["bk"].reshape(1, D),
                p["Wv"], p["bv"].reshape(1, D), p["Wo"], p["bo"].reshape(1, D),
                p["W1"], p["b1"].reshape(1, 1024), p["W2"], p["b2"].reshape(1, D),
                p["ln1_g"].reshape(1, D), p["ln1_b"].reshape(1, D),
                p["ln2_g"].reshape(1, D), p["ln2_b"].reshape(1, D))
    return out[:, :s_real]


# speedup vs baseline: 9.3687x; 2.8016x over previous
import functools
import math

import jax
import jax.numpy as jnp
from jax import lax
from jax.experimental import pallas as pl
from jax.experimental.pallas import tpu as pltpu
from jax.experimental.pallas import tpu_sc as plsc

D = 128
ACC_W = 144
NC, NS = 2, 16
NW = NC * NS
CHUNK = 32
SUB = 16
NSUB = CHUNK // SUB
ROW_BLK = 1280


def _edge_body(C, hs_hbm, hd_hbm, cidx_hbm, att_hbm,
               feat_hbm, s_hbm,
               acc_sh, ibuf, dbuf, hs_buf, hd_buf, out_stage, att_v, zrow,
               t_ref, w_ref, s_loc, sg0h, sg0d, sg1h, sg1d):
    npad = hs_hbm.shape[0]
    cid = lax.axis_index("c")
    sid = lax.axis_index("s")
    wid = sid * NC + cid

    pltpu.sync_copy(att_hbm, att_v)

    z16 = jnp.zeros((16,), jnp.float32)
    for r in range(SUB):
        for c in range(D // 16):
            zrow[r, pl.ds(c * 16, 16)] = z16

    rows_per_sub = npad // NS

    def _zero(k, carry):
        pltpu.sync_copy(zrow, acc_sh.at[pl.ds(sid * rows_per_sub + k * SUB, SUB)])
        return carry
    lax.fori_loop(0, rows_per_sub // SUB, _zero, 0)

    def _zero_s(k, carry):
        s_loc[pl.ds(k * 16, 16)] = z16
        return carry
    lax.fori_loop(0, npad // 16, _zero_s, 0)

    plsc.subcore_barrier()

    lanes = lax.iota(jnp.int32, SUB)
    g_h = (sg0h, sg1h)
    g_d = (sg0d, sg1d)

    pltpu.sync_copy(cidx_hbm.at[wid, pl.ds(0, 2)], ibuf)
    pltpu.sync_copy(hs_hbm.at[ibuf.at[0, 0]], hs_buf.at[0])
    pltpu.sync_copy(hd_hbm.at[ibuf.at[0, 1]], hd_buf.at[0])

    def _loop(i, carry):
        def _issue(b):
            return (pltpu.async_copy(hs_hbm.at[ibuf.at[b, 0]], hs_buf.at[b],
                                     g_h[b]),
                    pltpu.async_copy(hd_hbm.at[ibuf.at[b, 1]], hd_buf.at[b],
                                     g_d[b]))

        for b2 in range(2):
            for k in range(CHUNK // 16):
                dbuf[pl.ds(b2 * CHUNK + k * 16, 16)] = (
                    ibuf[b2, 1, pl.ds(k * 16, 16)])

        def _chunk_compute(b):
            def _sub(sb, cc):
                base = sb * SUB
                att_regs = [att_v[pl.ds(c * 16, 16)] for c in range(D // 16)]
                for e_i in range(SUB):
                    r = base + e_i
                    acc0 = None
                    acc1 = None
                    for c in range(D // 16):
                        a = hs_buf[b, r, pl.ds(c * 16, 16)]
                        t = a + hd_buf[b, r, pl.ds(c * 16, 16)]
                        zl = jnp.maximum(t, 0.2 * t)
                        term = zl * att_regs[c]
                        if c % 2 == 0:
                            acc0 = term if acc0 is None else acc0 + term
                        else:
                            acc1 = term if acc1 is None else acc1 + term
                    t_ref[e_i, :] = acc0 + acc1
                cols = [plsc.load_gather(t_ref,
                                         [lanes, jnp.full((SUB,), c, jnp.int32)])
                        for c in range(SUB)]
                while len(cols) > 1:
                    cols = [cols[j] + cols[j + 1] for j in range(0, len(cols), 2)]
                w_vec = jnp.exp(cols[0])
                w_ref[...] = w_vec
                dst_vec = dbuf[pl.ds(b * CHUNK + base, SUB)]
                for li in range(SUB):
                    plsc.addupdate_scatter(s_loc, [dst_vec], w_vec,
                                           mask=lanes == li)
                for e_i in range(SUB):
                    ws = plsc.load_gather(
                        w_ref, [jnp.full((SUB,), e_i, jnp.int32)])
                    r = base + e_i
                    ro = b * CHUNK + base + e_i
                    for c in range(D // 16):
                        out_stage[ro, pl.ds(c * 16, 16)] = (
                            hs_buf[b, r, pl.ds(c * 16, 16)] * ws)
                return cc
            lax.fori_loop(0, NSUB, _sub, 0)

        d_b = _issue(1)
        _chunk_compute(0)
        d_b[0].wait()
        d_b[1].wait()
        pltpu.sync_copy(cidx_hbm.at[wid, pl.ds(2 * i + 2, 2)], ibuf)
        d_a2 = _issue(0)
        _chunk_compute(1)
        pltpu.sync_copy(out_stage, acc_sh.at[dbuf], add=True)
        d_a2[0].wait()
        d_a2[1].wait()
        return carry
    lax.fori_loop(0, C // 2, _loop, 0)

    plsc.subcore_barrier()
    pltpu.sync_copy(acc_sh.at[pl.ds(sid * rows_per_sub, rows_per_sub)],
                    feat_hbm.at[cid, pl.ds(sid * rows_per_sub, rows_per_sub)])
    pltpu.sync_copy(s_loc, s_hbm.at[wid])


@functools.cache
def _make_edge_kernel(C, npad):
    mesh = plsc.VectorSubcoreMesh(core_axis_name="c", subcore_axis_name="s")
    return pl.kernel(
        functools.partial(_edge_body, C),
        out_type=(jax.ShapeDtypeStruct((NC, npad, D), jnp.float32),
                  jax.ShapeDtypeStruct((NW, npad), jnp.float32)),
        mesh=mesh,
        compiler_params=pltpu.CompilerParams(needs_layout_passes=False),
        scratch_types=[
            pltpu.VMEM_SHARED((npad, D), jnp.float32),
            pltpu.VMEM((2, 2, CHUNK), jnp.int32),
            pltpu.VMEM((2 * CHUNK,), jnp.int32),
            pltpu.VMEM((2, CHUNK, D), jnp.float32),
            pltpu.VMEM((2, CHUNK, D), jnp.float32),
            pltpu.VMEM((2 * CHUNK, D), jnp.float32),
            pltpu.VMEM((D,), jnp.float32),
            pltpu.VMEM((SUB, D), jnp.float32),
            pltpu.VMEM((SUB, SUB), jnp.float32),
            pltpu.VMEM((SUB,), jnp.float32),
            pltpu.VMEM((npad,), jnp.float32),
            pltpu.SemaphoreType.DMA,
            pltpu.SemaphoreType.DMA,
            pltpu.SemaphoreType.DMA,
            pltpu.SemaphoreType.DMA,
        ],
    )


def _prep_body(x_ref, wp_ref, bp_ref, ws_ref, wd_ref, h0_ref, hs_ref, hd_ref):
    h0 = jnp.dot(x_ref[...], wp_ref[...],
                 preferred_element_type=jnp.float32) + bp_ref[...]
    h0_ref[...] = h0
    hs_ref[...] = jnp.dot(h0, ws_ref[...], preferred_element_type=jnp.float32)
    hd_ref[...] = jnp.dot(h0, wd_ref[...], preferred_element_type=jnp.float32)


def _ln_rows(x, g, b):
    m = jnp.mean(x, axis=-1, keepdims=True)
    v = jnp.mean((x - m) ** 2, axis=-1, keepdims=True)
    return (x - m) / jnp.sqrt(v + 1e-5) * g + b


def _fin_mid_body(acc_ref, s_ref, bias_ref, g_ref, b_ref, ws_ref, wd_ref,
                  h_ref, hs_ref, hd_ref):
    a = acc_ref[0] + acc_ref[1]
    s = jnp.sum(s_ref[...], axis=0)[:, None]
    out = a / (s + 1e-30) + bias_ref[...]
    h = jnp.maximum(_ln_rows(out, g_ref[...], b_ref[...]), 0.0)
    h_ref[...] = h
    hs_ref[...] = jnp.dot(h, ws_ref[...], preferred_element_type=jnp.float32)
    hd_ref[...] = jnp.dot(h, wd_ref[...], preferred_element_type=jnp.float32)


def _fin_last_body(acc_ref, s_ref, bias_ref, h0_ref, wl_ref, bl_ref, bng_ref,
                   bnb_ref, xn_ref):
    a = acc_ref[0] + acc_ref[1]
    s = jnp.sum(s_ref[...], axis=0)[:, None]
    h = a / (s + 1e-30) + bias_ref[...]
    xn = jnp.dot(h0_ref[...] + h, wl_ref[...],
                 preferred_element_type=jnp.float32) + bl_ref[...]
    xn_ref[...] = xn / jnp.sqrt(1.0 + 1e-5) * bng_ref[...] + bnb_ref[...]


def _tr_body(seq_ref, mask_ref, wq_ref, bq_ref, wk_ref, bk_ref, wv_ref, bv_ref,
             wo_ref, bo_ref, w1_ref, b1_ref, w2_ref, b2_ref,
             g1_ref, gb1_ref, g2_ref, gb2_ref, out_ref):
    xx = seq_ref[0]
    q = jnp.dot(xx, wq_ref[...], preferred_element_type=jnp.float32) + bq_ref[...]
    k = jnp.dot(xx, wk_ref[...], preferred_element_type=jnp.float32) + bk_ref[...]
    v = jnp.dot(xx, wv_ref[...], preferred_element_type=jnp.float32) + bv_ref[...]
    dh = 32
    scale = 1.0 / math.sqrt(float(dh))
    outs = []
    for h in range(4):
        qh = q[:, h * dh:(h + 1) * dh]
        kh = k[:, h * dh:(h + 1) * dh]
        vh = v[:, h * dh:(h + 1) * dh]
        sc = lax.dot_general(qh, kh, (((1,), (1,)), ((), ())),
                             preferred_element_type=jnp.float32) * scale
        sc = sc + mask_ref[...]
        m = jnp.max(sc, axis=1, keepdims=True)
        p = jnp.exp(sc - m)
        p = p / jnp.sum(p, axis=1, keepdims=True)
        outs.append(jnp.dot(p, vh, preferred_element_type=jnp.float32))
    sa = jnp.concatenate(outs, axis=1)
    sa = jnp.dot(sa, wo_ref[...], preferred_element_type=jnp.float32) + bo_ref[...]
    s1 = _ln_rows(xx + sa, g1_ref[...], gb1_ref[...])
    ff = jnp.maximum(jnp.dot(s1, w1_ref[...],
                             preferred_element_type=jnp.float32) + b1_ref[...], 0.0)
    ff = jnp.dot(ff, w2_ref[...], preferred_element_type=jnp.float32) + b2_ref[...]
    out_ref[0] = _ln_rows(s1 + ff, g2_ref[...], gb2_ref[...])


def _row_kernel(body, npad, n_out, extra_specs, grid_rows=ROW_BLK):
    ngrid = npad // grid_rows
    rb = lambda i: (i, 0)
    full2 = lambda i: (0, 0)
    return pl.pallas_call(
        body,
        grid=(ngrid,),
        in_specs=extra_specs,
        out_specs=[pl.BlockSpec((grid_rows, D), rb)] * n_out,
        out_shape=[jax.ShapeDtypeStruct((npad, D), jnp.float32)] * n_out,
    )


def kernel(x, edge_index, ptr, params):
    p = params
    n = x.shape[0]
    npad = ((n + NS * SUB * 40 - 1) // (NS * SUB * 40)) * (NS * SUB * 40)
    b_graphs = ptr.shape[0] - 1
    seq_len = n // b_graphs

    xp = jnp.zeros((npad, D), jnp.float32).at[:n].set(x)
    loop = jnp.arange(n, dtype=edge_index.dtype)
    e_all = edge_index.shape[1] + n
    c_chunks = -(-e_all // (NW * CHUNK))
    e_pad = NW * CHUNK * c_chunks
    fill = jnp.full((e_pad - e_all,), n, edge_index.dtype)
    src_r = jnp.concatenate([edge_index[0], loop, fill]).reshape(NW, c_chunks, CHUNK)
    dst_r = jnp.concatenate([edge_index[1], loop, fill]).reshape(NW, c_chunks, CHUNK)
    cidx = jnp.stack([src_r, dst_r], axis=2)
    cidx = jnp.concatenate(
        [cidx, jnp.full((NW, 2, 2, CHUNK), n, edge_index.dtype)], axis=1)

    rb = lambda i: (i, 0)
    w_spec = pl.BlockSpec((D, D), lambda i: (0, 0))
    b_spec = pl.BlockSpec((1, D), lambda i: (0, 0))
    row_spec = pl.BlockSpec((ROW_BLK, D), rb)
    acc_spec = pl.BlockSpec((NC, ROW_BLK, D), lambda i: (0, i, 0))
    s_spec = pl.BlockSpec((NW, ROW_BLK), lambda i: (0, i))

    prep = _row_kernel(_prep_body, npad, 3,
                       [row_spec, w_spec, b_spec, w_spec, w_spec])
    h0, hs, hd = prep(xp, p["W_proj"], p["b_proj"].reshape(1, D),
                      p["gat_W_src"][0], p["gat_W_dst"][0])

    edge_k = _make_edge_kernel(c_chunks, npad)
    nl = p["gat_W_src"].shape[0]

    fin_mid = pl.pallas_call(
        _fin_mid_body,
        grid=(npad // ROW_BLK,),
        in_specs=[acc_spec, s_spec, b_spec, b_spec, b_spec, w_spec, w_spec],
        out_specs=[row_spec] * 3,
        out_shape=[jax.ShapeDtypeStruct((npad, D), jnp.float32)] * 3,
    )
    for l in range(nl - 1):
        acc, svec = edge_k(hs, hd, cidx, p["gat_att"][l])
        h, hs, hd = fin_mid(acc, svec, p["gat_bias"][l].reshape(1, D),
                            p["gat_ln_g"][l].reshape(1, D),
                            p["gat_ln_b"][l].reshape(1, D),
                            p["gat_W_src"][l + 1], p["gat_W_dst"][l + 1])

    acc, svec = edge_k(hs, hd, cidx, p["gat_att"][nl - 1])
    fin_last = pl.pallas_call(
        _fin_last_body,
        grid=(npad // ROW_BLK,),
        in_specs=[acc_spec, s_spec, b_spec, row_spec, w_spec, b_spec, b_spec,
                  b_spec],
        out_specs=[row_spec],
        out_shape=[jax.ShapeDtypeStruct((npad, D), jnp.float32)],
    )
    (xn,) = fin_last(acc, svec, p["gat_bias"][nl - 1].reshape(1, D), h0,
                     p["W_lin"], p["b_lin"].reshape(1, D),
                     p["bn_g"].reshape(1, D), p["bn_b"].reshape(1, D))

    X = xn[:n].reshape(b_graphs, seq_len, D)
    tok = lambda t: jnp.tile(t[None, None, :], (b_graphs, 1, 1))
    seq = jnp.concatenate([tok(p["CLS"]), X, tok(p["RING"]), tok(p["END"])],
                          axis=1)
    s_real = seq.shape[1]
    s_pad = 128
    seqp = jnp.zeros((b_graphs, s_pad, D), jnp.float32).at[:, :s_real].set(seq)
    mask = jnp.where(jnp.arange(s_pad) < s_real, 0.0, -1e30)
    mask = mask.astype(jnp.float32).reshape(1, s_pad)

    sblk = pl.BlockSpec((1, s_pad, D), lambda i: (i, 0, 0))
    full2 = lambda i: (0, 0)
    wspec = pl.BlockSpec((D, D), full2)
    bspec = pl.BlockSpec((1, D), full2)
    mspec = pl.BlockSpec((1, s_pad), full2)
    w1spec = pl.BlockSpec((D, 1024), full2)
    b1spec = pl.BlockSpec((1, 1024), full2)
    w2spec = pl.BlockSpec((1024, D), full2)
    tr = pl.pallas_call(
        _tr_body,
        grid=(b_graphs,),
        in_specs=[sblk, mspec,
                  wspec, bspec, wspec, bspec, wspec, bspec, wspec, bspec,
                  w1spec, b1spec, w2spec, bspec,
                  bspec, bspec, bspec, bspec],
        out_specs=[sblk],
        out_shape=[jax.ShapeDtypeStruct((b_graphs, s_pad, D), jnp.float32)],
    )
    (out,) = tr(seqp, mask,
                p["Wq"], p["bq"].reshape(1, D), p["Wk"], p["bk"].reshape(1, D),
                p["Wv"], p["bv"].reshape(1, D), p["Wo"], p["bo"].reshape(1, D),
                p["W1"], p["b1"].reshape(1, 1024), p["W2"], p["b2"].reshape(1, D),
                p["ln1_g"].reshape(1, D), p["ln1_b"].reshape(1, D),
                p["ln2_g"].reshape(1, D), p["ln2_b"].reshape(1, D))
    return out[:, :s_real]


# speedup vs baseline: 9.6887x; 1.0342x over previous
import functools
import math

import jax
import jax.numpy as jnp
from jax import lax
from jax.experimental import pallas as pl
from jax.experimental.pallas import tpu as pltpu
from jax.experimental.pallas import tpu_sc as plsc

D = 128
ACC_W = 144
NC, NS = 2, 16
NW = NC * NS
CHUNK = 32
SUB = 16
NSUB = CHUNK // SUB
ROW_BLK = 1280


def _edge_body(C, hs_hbm, hd_hbm, cidx_hbm, att_hbm,
               feat_hbm, s_hbm,
               acc_sh, ibuf, dbuf, hs_buf, hd_buf, out_stage, att_v, zrow,
               t_ref, w_ref, s_loc, sg0h, sg0d, sg1h, sg1d, ssc):
    npad = hs_hbm.shape[0]
    cid = lax.axis_index("c")
    sid = lax.axis_index("s")
    wid = sid * NC + cid

    pltpu.sync_copy(att_hbm, att_v)

    z16 = jnp.zeros((16,), jnp.float32)
    for r in range(SUB):
        for c in range(D // 16):
            zrow[r, pl.ds(c * 16, 16)] = z16

    rows_per_sub = npad // NS

    def _zero(k, carry):
        pltpu.sync_copy(zrow, acc_sh.at[pl.ds(sid * rows_per_sub + k * SUB, SUB)])
        return carry
    lax.fori_loop(0, rows_per_sub // SUB, _zero, 0)

    def _zero_s(k, carry):
        s_loc[pl.ds(k * 16, 16)] = z16
        return carry
    lax.fori_loop(0, npad // 16, _zero_s, 0)

    z16i = jnp.zeros((16,), jnp.int32)
    for k in range(2 * CHUNK // 16):
        dbuf[pl.ds(k * 16, 16)] = z16i

    def _zero_o(r2, carry):
        for c in range(D // 16):
            out_stage[r2, pl.ds(c * 16, 16)] = z16
        return carry
    lax.fori_loop(0, 2 * CHUNK, _zero_o, 0)

    plsc.subcore_barrier()

    lanes = lax.iota(jnp.int32, SUB)
    g_h = (sg0h, sg1h)
    g_d = (sg0d, sg1d)

    pltpu.async_copy(out_stage, acc_sh.at[dbuf], ssc, add=True)

    pltpu.sync_copy(cidx_hbm.at[wid, pl.ds(0, 2)], ibuf)
    pltpu.sync_copy(hs_hbm.at[ibuf.at[0, 0]], hs_buf.at[0])
    pltpu.sync_copy(hd_hbm.at[ibuf.at[0, 1]], hd_buf.at[0])

    def _loop(i, carry):
        def _issue(b):
            return (pltpu.async_copy(hs_hbm.at[ibuf.at[b, 0]], hs_buf.at[b],
                                     g_h[b]),
                    pltpu.async_copy(hd_hbm.at[ibuf.at[b, 1]], hd_buf.at[b],
                                     g_d[b]))

        pltpu.make_async_copy(out_stage, acc_sh.at[dbuf], ssc).wait()

        for b2 in range(2):
            for k in range(CHUNK // 16):
                dbuf[pl.ds(b2 * CHUNK + k * 16, 16)] = (
                    ibuf[b2, 1, pl.ds(k * 16, 16)])

        def _chunk_compute(b):
            def _sub(sb, cc):
                base = sb * SUB
                att_regs = [att_v[pl.ds(c * 16, 16)] for c in range(D // 16)]
                for e_i in range(SUB):
                    r = base + e_i
                    acc0 = None
                    acc1 = None
                    for c in range(D // 16):
                        a = hs_buf[b, r, pl.ds(c * 16, 16)]
                        t = a + hd_buf[b, r, pl.ds(c * 16, 16)]
                        zl = jnp.maximum(t, 0.2 * t)
                        term = zl * att_regs[c]
                        if c % 2 == 0:
                            acc0 = term if acc0 is None else acc0 + term
                        else:
                            acc1 = term if acc1 is None else acc1 + term
                    t_ref[e_i, :] = acc0 + acc1
                cols = [plsc.load_gather(t_ref,
                                         [lanes, jnp.full((SUB,), c, jnp.int32)])
                        for c in range(SUB)]
                while len(cols) > 1:
                    cols = [cols[j] + cols[j + 1] for j in range(0, len(cols), 2)]
                w_vec = jnp.exp(cols[0])
                w_ref[...] = w_vec
                dst_vec = dbuf[pl.ds(b * CHUNK + base, SUB)]
                for li in range(SUB):
                    plsc.addupdate_scatter(s_loc, [dst_vec], w_vec,
                                           mask=lanes == li)
                for e_i in range(SUB):
                    ws = plsc.load_gather(
                        w_ref, [jnp.full((SUB,), e_i, jnp.int32)])
                    r = base + e_i
                    ro = b * CHUNK + base + e_i
                    for c in range(D // 16):
                        out_stage[ro, pl.ds(c * 16, 16)] = (
                            hs_buf[b, r, pl.ds(c * 16, 16)] * ws)
                return cc
            lax.fori_loop(0, NSUB, _sub, 0)

        d_b = _issue(1)
        _chunk_compute(0)
        d_b[0].wait()
        d_b[1].wait()
        pltpu.sync_copy(cidx_hbm.at[wid, pl.ds(2 * i + 2, 2)], ibuf)
        d_a2 = _issue(0)
        _chunk_compute(1)
        pltpu.async_copy(out_stage, acc_sh.at[dbuf], ssc, add=True)
        d_a2[0].wait()
        d_a2[1].wait()
        return carry
    lax.fori_loop(0, C // 2, _loop, 0)

    pltpu.make_async_copy(out_stage, acc_sh.at[dbuf], ssc).wait()
    plsc.subcore_barrier()
    pltpu.sync_copy(acc_sh.at[pl.ds(sid * rows_per_sub, rows_per_sub)],
                    feat_hbm.at[cid, pl.ds(sid * rows_per_sub, rows_per_sub)])
    pltpu.sync_copy(s_loc, s_hbm.at[wid])


@functools.cache
def _make_edge_kernel(C, npad):
    mesh = plsc.VectorSubcoreMesh(core_axis_name="c", subcore_axis_name="s")
    return pl.kernel(
        functools.partial(_edge_body, C),
        out_type=(jax.ShapeDtypeStruct((NC, npad, D), jnp.float32),
                  jax.ShapeDtypeStruct((NW, npad), jnp.float32)),
        mesh=mesh,
        compiler_params=pltpu.CompilerParams(needs_layout_passes=False),
        scratch_types=[
            pltpu.VMEM_SHARED((npad, D), jnp.float32),
            pltpu.VMEM((2, 2, CHUNK), jnp.int32),
            pltpu.VMEM((2 * CHUNK,), jnp.int32),
            pltpu.VMEM((2, CHUNK, D), jnp.float32),
            pltpu.VMEM((2, CHUNK, D), jnp.float32),
            pltpu.VMEM((2 * CHUNK, D), jnp.float32),
            pltpu.VMEM((D,), jnp.float32),
            pltpu.VMEM((SUB, D), jnp.float32),
            pltpu.VMEM((SUB, SUB), jnp.float32),
            pltpu.VMEM((SUB,), jnp.float32),
            pltpu.VMEM((npad,), jnp.float32),
            pltpu.SemaphoreType.DMA,
            pltpu.SemaphoreType.DMA,
            pltpu.SemaphoreType.DMA,
            pltpu.SemaphoreType.DMA,
            pltpu.SemaphoreType.DMA,
        ],
    )


def _prep_body(x_ref, wp_ref, bp_ref, ws_ref, wd_ref, h0_ref, hs_ref, hd_ref):
    h0 = jnp.dot(x_ref[...], wp_ref[...],
                 preferred_element_type=jnp.float32) + bp_ref[...]
    h0_ref[...] = h0
    hs_ref[...] = jnp.dot(h0, ws_ref[...], preferred_element_type=jnp.float32)
    hd_ref[...] = jnp.dot(h0, wd_ref[...], preferred_element_type=jnp.float32)


def _ln_rows(x, g, b):
    m = jnp.mean(x, axis=-1, keepdims=True)
    v = jnp.mean((x - m) ** 2, axis=-1, keepdims=True)
    return (x - m) / jnp.sqrt(v + 1e-5) * g + b


def _fin_mid_body(acc_ref, s_ref, bias_ref, g_ref, b_ref, ws_ref, wd_ref,
                  h_ref, hs_ref, hd_ref):
    a = acc_ref[0] + acc_ref[1]
    s = jnp.sum(s_ref[...], axis=0)[:, None]
    out = a / (s + 1e-30) + bias_ref[...]
    h = jnp.maximum(_ln_rows(out, g_ref[...], b_ref[...]), 0.0)
    h_ref[...] = h
    hs_ref[...] = jnp.dot(h, ws_ref[...], preferred_element_type=jnp.float32)
    hd_ref[...] = jnp.dot(h, wd_ref[...], preferred_element_type=jnp.float32)


def _fin_last_body(acc_ref, s_ref, bias_ref, h0_ref, wl_ref, bl_ref, bng_ref,
                   bnb_ref, xn_ref):
    a = acc_ref[0] + acc_ref[1]
    s = jnp.sum(s_ref[...], axis=0)[:, None]
    h = a / (s + 1e-30) + bias_ref[...]
    xn = jnp.dot(h0_ref[...] + h, wl_ref[...],
                 preferred_element_type=jnp.float32) + bl_ref[...]
    xn_ref[...] = xn / jnp.sqrt(1.0 + 1e-5) * bng_ref[...] + bnb_ref[...]


def _tr_body(seq_ref, mask_ref, wq_ref, bq_ref, wk_ref, bk_ref, wv_ref, bv_ref,
             wo_ref, bo_ref, w1_ref, b1_ref, w2_ref, b2_ref,
             g1_ref, gb1_ref, g2_ref, gb2_ref, out_ref):
    xx = seq_ref[0]
    q = jnp.dot(xx, wq_ref[...], preferred_element_type=jnp.float32) + bq_ref[...]
    k = jnp.dot(xx, wk_ref[...], preferred_element_type=jnp.float32) + bk_ref[...]
    v = jnp.dot(xx, wv_ref[...], preferred_element_type=jnp.float32) + bv_ref[...]
    dh = 32
    scale = 1.0 / math.sqrt(float(dh))
    outs = []
    for h in range(4):
        qh = q[:, h * dh:(h + 1) * dh]
        kh = k[:, h * dh:(h + 1) * dh]
        vh = v[:, h * dh:(h + 1) * dh]
        sc = lax.dot_general(qh, kh, (((1,), (1,)), ((), ())),
                             preferred_element_type=jnp.float32) * scale
        sc = sc + mask_ref[...]
        m = jnp.max(sc, axis=1, keepdims=True)
        p = jnp.exp(sc - m)
        p = p / jnp.sum(p, axis=1, keepdims=True)
        outs.append(jnp.dot(p, vh, preferred_element_type=jnp.float32))
    sa = jnp.concatenate(outs, axis=1)
    sa = jnp.dot(sa, wo_ref[...], preferred_element_type=jnp.float32) + bo_ref[...]
    s1 = _ln_rows(xx + sa, g1_ref[...], gb1_ref[...])
    ff = jnp.maximum(jnp.dot(s1, w1_ref[...],
                             preferred_element_type=jnp.float32) + b1_ref[...], 0.0)
    ff = jnp.dot(ff, w2_ref[...], preferred_element_type=jnp.float32) + b2_ref[...]
    out_ref[0] = _ln_rows(s1 + ff, g2_ref[...], gb2_ref[...])


def _row_kernel(body, npad, n_out, extra_specs, grid_rows=ROW_BLK):
    ngrid = npad // grid_rows
    rb = lambda i: (i, 0)
    full2 = lambda i: (0, 0)
    return pl.pallas_call(
        body,
        grid=(ngrid,),
        in_specs=extra_specs,
        out_specs=[pl.BlockSpec((grid_rows, D), rb)] * n_out,
        out_shape=[jax.ShapeDtypeStruct((npad, D), jnp.float32)] * n_out,
    )


def kernel(x, edge_index, ptr, params):
    p = params
    n = x.shape[0]
    npad = ((n + NS * SUB * 40 - 1) // (NS * SUB * 40)) * (NS * SUB * 40)
    b_graphs = ptr.shape[0] - 1
    seq_len = n // b_graphs

    xp = jnp.zeros((npad, D), jnp.float32).at[:n].set(x)
    loop = jnp.arange(n, dtype=edge_index.dtype)
    e_all = edge_index.shape[1] + n
    c_chunks = -(-e_all // (NW * CHUNK))
    e_pad = NW * CHUNK * c_chunks
    fill = jnp.full((e_pad - e_all,), n, edge_index.dtype)
    src_r = jnp.concatenate([edge_index[0], loop, fill]).reshape(NW, c_chunks, CHUNK)
    dst_r = jnp.concatenate([edge_index[1], loop, fill]).reshape(NW, c_chunks, CHUNK)
    cidx = jnp.stack([src_r, dst_r], axis=2)
    cidx = jnp.concatenate(
        [cidx, jnp.full((NW, 2, 2, CHUNK), n, edge_index.dtype)], axis=1)

    rb = lambda i: (i, 0)
    w_spec = pl.BlockSpec((D, D), lambda i: (0, 0))
    b_spec = pl.BlockSpec((1, D), lambda i: (0, 0))
    row_spec = pl.BlockSpec((ROW_BLK, D), rb)
    acc_spec = pl.BlockSpec((NC, ROW_BLK, D), lambda i: (0, i, 0))
    s_spec = pl.BlockSpec((NW, ROW_BLK), lambda i: (0, i))

    prep = _row_kernel(_prep_body, npad, 3,
                       [row_spec, w_spec, b_spec, w_spec, w_spec])
    h0, hs, hd = prep(xp, p["W_proj"], p["b_proj"].reshape(1, D),
                      p["gat_W_src"][0], p["gat_W_dst"][0])

    edge_k = _make_edge_kernel(c_chunks, npad)
    nl = p["gat_W_src"].shape[0]

    fin_mid = pl.pallas_call(
        _fin_mid_body,
        grid=(npad // ROW_BLK,),
        in_specs=[acc_spec, s_spec, b_spec, b_spec, b_spec, w_spec, w_spec],
        out_specs=[row_spec] * 3,
        out_shape=[jax.ShapeDtypeStruct((npad, D), jnp.float32)] * 3,
    )
    for l in range(nl - 1):
        acc, svec = edge_k(hs, hd, cidx, p["gat_att"][l])
        h, hs, hd = fin_mid(acc, svec, p["gat_bias"][l].reshape(1, D),
                            p["gat_ln_g"][l].reshape(1, D),
                            p["gat_ln_b"][l].reshape(1, D),
                            p["gat_W_src"][l + 1], p["gat_W_dst"][l + 1])

    acc, svec = edge_k(hs, hd, cidx, p["gat_att"][nl - 1])
    fin_last = pl.pallas_call(
        _fin_last_body,
        grid=(npad // ROW_BLK,),
        in_specs=[acc_spec, s_spec, b_spec, row_spec, w_spec, b_spec, b_spec,
                  b_spec],
        out_specs=[row_spec],
        out_shape=[jax.ShapeDtypeStruct((npad, D), jnp.float32)],
    )
    (xn,) = fin_last(acc, svec, p["gat_bias"][nl - 1].reshape(1, D), h0,
                     p["W_lin"], p["b_lin"].reshape(1, D),
                     p["bn_g"].reshape(1, D), p["bn_b"].reshape(1, D))

    X = xn[:n].reshape(b_graphs, seq_len, D)
    tok = lambda t: jnp.tile(t[None, None, :], (b_graphs, 1, 1))
    seq = jnp.concatenate([tok(p["CLS"]), X, tok(p["RING"]), tok(p["END"])],
                          axis=1)
    s_real = seq.shape[1]
    s_pad = 128
    seqp = jnp.zeros((b_graphs, s_pad, D), jnp.float32).at[:, :s_real].set(seq)
    mask = jnp.where(jnp.arange(s_pad) < s_real, 0.0, -1e30)
    mask = mask.astype(jnp.float32).reshape(1, s_pad)

    sblk = pl.BlockSpec((1, s_pad, D), lambda i: (i, 0, 0))
    full2 = lambda i: (0, 0)
    wspec = pl.BlockSpec((D, D), full2)
    bspec = pl.BlockSpec((1, D), full2)
    mspec = pl.BlockSpec((1, s_pad), full2)
    w1spec = pl.BlockSpec((D, 1024), full2)
    b1spec = pl.BlockSpec((1, 1024), full2)
    w2spec = pl.BlockSpec((1024, D), full2)
    tr = pl.pallas_call(
        _tr_body,
        grid=(b_graphs,),
        in_specs=[sblk, mspec,
                  wspec, bspec, wspec, bspec, wspec, bspec, wspec, bspec,
                  w1spec, b1spec, w2spec, bspec,
                  bspec, bspec, bspec, bspec],
        out_specs=[sblk],
        out_shape=[jax.ShapeDtypeStruct((b_graphs, s_pad, D), jnp.float32)],
    )
    (out,) = tr(seqp, mask,
                p["Wq"], p["bq"].reshape(1, D), p["Wk"], p["bk"].reshape(1, D),
                p["Wv"], p["bv"].reshape(1, D), p["Wo"], p["bo"].reshape(1, D),
                p["W1"], p["b1"].reshape(1, 1024), p["W2"], p["b2"].reshape(1, D),
                p["ln1_g"].reshape(1, D), p["ln1_b"].reshape(1, D),
                p["ln2_g"].reshape(1, D), p["ln2_b"].reshape(1, D))
    return out[:, :s_real]


# speedup vs baseline: 10.4634x; 1.0800x over previous
import functools
import math

import jax
import jax.numpy as jnp
from jax import lax
from jax.experimental import pallas as pl
from jax.experimental.pallas import tpu as pltpu
from jax.experimental.pallas import tpu_sc as plsc

D = 128
ACC_W = 144
NC, NS = 2, 16
NW = NC * NS
CHUNK = 32
SUB = 16
NSUB = CHUNK // SUB
ROW_BLK = 1280


def _edge_body(C, hs_hbm, hd_hbm, cidx_hbm, att_hbm,
               feat_hbm, s_hbm,
               acc_sh, ibuf, ibuf2, dbuf, hs_buf, hd_buf, out_stage, att_v,
               zrow, t_ref, w_ref, s_loc, sg0h, sg0d, sg1h, sg1d, ssc, sidx):
    npad = hs_hbm.shape[0]
    cid = lax.axis_index("c")
    sid = lax.axis_index("s")
    wid = sid * NC + cid

    pltpu.sync_copy(att_hbm, att_v)

    z16 = jnp.zeros((16,), jnp.float32)
    for r in range(SUB):
        for c in range(D // 16):
            zrow[r, pl.ds(c * 16, 16)] = z16

    rows_per_sub = npad // NS

    def _zero(k, carry):
        pltpu.sync_copy(zrow, acc_sh.at[pl.ds(sid * rows_per_sub + k * SUB, SUB)])
        return carry
    lax.fori_loop(0, rows_per_sub // SUB, _zero, 0)

    def _zero_s(k, carry):
        s_loc[pl.ds(k * 16, 16)] = z16
        return carry
    lax.fori_loop(0, npad // 16, _zero_s, 0)

    z16i = jnp.zeros((16,), jnp.int32)
    for k in range(2 * CHUNK // 16):
        dbuf[pl.ds(k * 16, 16)] = z16i

    def _zero_o(r2, carry):
        for c in range(D // 16):
            out_stage[r2, pl.ds(c * 16, 16)] = z16
        return carry
    lax.fori_loop(0, 2 * CHUNK, _zero_o, 0)

    plsc.subcore_barrier()

    lanes = lax.iota(jnp.int32, SUB)
    g_h = (sg0h, sg1h)
    g_d = (sg0d, sg1d)

    pltpu.async_copy(out_stage, acc_sh.at[dbuf], ssc, add=True)

    pltpu.sync_copy(cidx_hbm.at[wid, pl.ds(0, 2)], ibuf)
    pltpu.sync_copy(hs_hbm.at[ibuf.at[0, 0]], hs_buf.at[0])
    pltpu.sync_copy(hd_hbm.at[ibuf.at[0, 1]], hd_buf.at[0])

    def _pair(i, ib_cur, ib_next):
        def _issue(ib, b, into):
            return (pltpu.async_copy(hs_hbm.at[ib.at[b, 0]], hs_buf.at[into],
                                     g_h[into]),
                    pltpu.async_copy(hd_hbm.at[ib.at[b, 1]], hd_buf.at[into],
                                     g_d[into]))

        d_b = _issue(ib_cur, 1, 1)

        pltpu.make_async_copy(out_stage, acc_sh.at[dbuf], ssc).wait()

        for b2 in range(2):
            for k in range(CHUNK // 16):
                dbuf[pl.ds(b2 * CHUNK + k * 16, 16)] = (
                    ib_cur[b2, 1, pl.ds(k * 16, 16)])

        d_i = pltpu.async_copy(cidx_hbm.at[wid, pl.ds(2 * i + 2, 2)],
                               ib_next, sidx)

        def _chunk_compute(b):
            def _sub(sb, cc):
                base = sb * SUB
                att_regs = [att_v[pl.ds(c * 16, 16)] for c in range(D // 16)]
                for e_i in range(SUB):
                    r = base + e_i
                    acc0 = None
                    acc1 = None
                    for c in range(D // 16):
                        a = hs_buf[b, r, pl.ds(c * 16, 16)]
                        t = a + hd_buf[b, r, pl.ds(c * 16, 16)]
                        zl = jnp.maximum(t, 0.2 * t)
                        term = zl * att_regs[c]
                        if c % 2 == 0:
                            acc0 = term if acc0 is None else acc0 + term
                        else:
                            acc1 = term if acc1 is None else acc1 + term
                    t_ref[e_i, :] = acc0 + acc1
                cols = [plsc.load_gather(t_ref,
                                         [lanes, jnp.full((SUB,), c, jnp.int32)])
                        for c in range(SUB)]
                while len(cols) > 1:
                    cols = [cols[j] + cols[j + 1] for j in range(0, len(cols), 2)]
                w_vec = jnp.exp(cols[0])
                w_ref[...] = w_vec
                dst_vec = dbuf[pl.ds(b * CHUNK + base, SUB)]
                for li in range(SUB):
                    plsc.addupdate_scatter(s_loc, [dst_vec], w_vec,
                                           mask=lanes == li)
                for e_i in range(SUB):
                    ws = plsc.load_gather(
                        w_ref, [jnp.full((SUB,), e_i, jnp.int32)])
                    r = base + e_i
                    ro = b * CHUNK + base + e_i
                    for c in range(D // 16):
                        out_stage[ro, pl.ds(c * 16, 16)] = (
                            hs_buf[b, r, pl.ds(c * 16, 16)] * ws)
                return cc
            lax.fori_loop(0, NSUB, _sub, 0)

        _chunk_compute(0)
        d_b[0].wait()
        d_b[1].wait()
        d_i.wait()
        d_a2 = _issue(ib_next, 0, 0)
        _chunk_compute(1)
        pltpu.async_copy(out_stage, acc_sh.at[dbuf], ssc, add=True)
        d_a2[0].wait()
        d_a2[1].wait()

    def _loop(j, carry):
        _pair(2 * j, ibuf, ibuf2)
        _pair(2 * j + 1, ibuf2, ibuf)
        return carry
    lax.fori_loop(0, C // 4, _loop, 0)

    pltpu.make_async_copy(out_stage, acc_sh.at[dbuf], ssc).wait()
    plsc.subcore_barrier()
    pltpu.sync_copy(acc_sh.at[pl.ds(sid * rows_per_sub, rows_per_sub)],
                    feat_hbm.at[cid, pl.ds(sid * rows_per_sub, rows_per_sub)])
    pltpu.sync_copy(s_loc, s_hbm.at[wid])


@functools.cache
def _make_edge_kernel(C, npad):
    mesh = plsc.VectorSubcoreMesh(core_axis_name="c", subcore_axis_name="s")
    return pl.kernel(
        functools.partial(_edge_body, C),
        out_type=(jax.ShapeDtypeStruct((NC, npad, D), jnp.float32),
                  jax.ShapeDtypeStruct((NW, npad), jnp.float32)),
        mesh=mesh,
        compiler_params=pltpu.CompilerParams(needs_layout_passes=False),
        scratch_types=[
            pltpu.VMEM_SHARED((npad, D), jnp.float32),
            pltpu.VMEM((2, 2, CHUNK), jnp.int32),
            pltpu.VMEM((2, 2, CHUNK), jnp.int32),
            pltpu.VMEM((2 * CHUNK,), jnp.int32),
            pltpu.VMEM((2, CHUNK, D), jnp.float32),
            pltpu.VMEM((2, CHUNK, D), jnp.float32),
            pltpu.VMEM((2 * CHUNK, D), jnp.float32),
            pltpu.VMEM((D,), jnp.float32),
            pltpu.VMEM((SUB, D), jnp.float32),
            pltpu.VMEM((SUB, SUB), jnp.float32),
            pltpu.VMEM((SUB,), jnp.float32),
            pltpu.VMEM((npad,), jnp.float32),
            pltpu.SemaphoreType.DMA,
            pltpu.SemaphoreType.DMA,
            pltpu.SemaphoreType.DMA,
            pltpu.SemaphoreType.DMA,
            pltpu.SemaphoreType.DMA,
            pltpu.SemaphoreType.DMA,
        ],
    )


def _prep_body(x_ref, wp_ref, bp_ref, ws_ref, wd_ref, h0_ref, hs_ref, hd_ref):
    h0 = jnp.dot(x_ref[...], wp_ref[...],
                 preferred_element_type=jnp.float32) + bp_ref[...]
    h0_ref[...] = h0
    hs_ref[...] = jnp.dot(h0, ws_ref[...], preferred_element_type=jnp.float32)
    hd_ref[...] = jnp.dot(h0, wd_ref[...], preferred_element_type=jnp.float32)


def _ln_rows(x, g, b):
    m = jnp.mean(x, axis=-1, keepdims=True)
    v = jnp.mean((x - m) ** 2, axis=-1, keepdims=True)
    return (x - m) / jnp.sqrt(v + 1e-5) * g + b


def _fin_mid_body(acc_ref, s_ref, bias_ref, g_ref, b_ref, ws_ref, wd_ref,
                  h_ref, hs_ref, hd_ref):
    a = acc_ref[0] + acc_ref[1]
    s = jnp.sum(s_ref[...], axis=0)[:, None]
    out = a / (s + 1e-30) + bias_ref[...]
    h = jnp.maximum(_ln_rows(out, g_ref[...], b_ref[...]), 0.0)
    h_ref[...] = h
    hs_ref[...] = jnp.dot(h, ws_ref[...], preferred_element_type=jnp.float32)
    hd_ref[...] = jnp.dot(h, wd_ref[...], preferred_element_type=jnp.float32)


def _fin_last_body(acc_ref, s_ref, bias_ref, h0_ref, wl_ref, bl_ref, bng_ref,
                   bnb_ref, xn_ref):
    a = acc_ref[0] + acc_ref[1]
    s = jnp.sum(s_ref[...], axis=0)[:, None]
    h = a / (s + 1e-30) + bias_ref[...]
    xn = jnp.dot(h0_ref[...] + h, wl_ref[...],
                 preferred_element_type=jnp.float32) + bl_ref[...]
    xn_ref[...] = xn / jnp.sqrt(1.0 + 1e-5) * bng_ref[...] + bnb_ref[...]


def _tr_body(seq_ref, mask_ref, wq_ref, bq_ref, wk_ref, bk_ref, wv_ref, bv_ref,
             wo_ref, bo_ref, w1_ref, b1_ref, w2_ref, b2_ref,
             g1_ref, gb1_ref, g2_ref, gb2_ref, out_ref):
    xx = seq_ref[0]
    q = jnp.dot(xx, wq_ref[...], preferred_element_type=jnp.float32) + bq_ref[...]
    k = jnp.dot(xx, wk_ref[...], preferred_element_type=jnp.float32) + bk_ref[...]
    v = jnp.dot(xx, wv_ref[...], preferred_element_type=jnp.float32) + bv_ref[...]
    dh = 32
    scale = 1.0 / math.sqrt(float(dh))
    outs = []
    for h in range(4):
        qh = q[:, h * dh:(h + 1) * dh]
        kh = k[:, h * dh:(h + 1) * dh]
        vh = v[:, h * dh:(h + 1) * dh]
        sc = lax.dot_general(qh, kh, (((1,), (1,)), ((), ())),
                             preferred_element_type=jnp.float32) * scale
        sc = sc + mask_ref[...]
        m = jnp.max(sc, axis=1, keepdims=True)
        p = jnp.exp(sc - m)
        p = p / jnp.sum(p, axis=1, keepdims=True)
        outs.append(jnp.dot(p, vh, preferred_element_type=jnp.float32))
    sa = jnp.concatenate(outs, axis=1)
    sa = jnp.dot(sa, wo_ref[...], preferred_element_type=jnp.float32) + bo_ref[...]
    s1 = _ln_rows(xx + sa, g1_ref[...], gb1_ref[...])
    ff = jnp.maximum(jnp.dot(s1, w1_ref[...],
                             preferred_element_type=jnp.float32) + b1_ref[...], 0.0)
    ff = jnp.dot(ff, w2_ref[...], preferred_element_type=jnp.float32) + b2_ref[...]
    out_ref[0] = _ln_rows(s1 + ff, g2_ref[...], gb2_ref[...])


def _row_kernel(body, npad, n_out, extra_specs, grid_rows=ROW_BLK):
    ngrid = npad // grid_rows
    rb = lambda i: (i, 0)
    full2 = lambda i: (0, 0)
    return pl.pallas_call(
        body,
        grid=(ngrid,),
        in_specs=extra_specs,
        out_specs=[pl.BlockSpec((grid_rows, D), rb)] * n_out,
        out_shape=[jax.ShapeDtypeStruct((npad, D), jnp.float32)] * n_out,
    )


def kernel(x, edge_index, ptr, params):
    p = params
    n = x.shape[0]
    npad = ((n + NS * SUB * 40 - 1) // (NS * SUB * 40)) * (NS * SUB * 40)
    b_graphs = ptr.shape[0] - 1
    seq_len = n // b_graphs

    xp = jnp.zeros((npad, D), jnp.float32).at[:n].set(x)
    loop = jnp.arange(n, dtype=edge_index.dtype)
    e_all = edge_index.shape[1] + n
    c_chunks = -(-e_all // (NW * CHUNK))
    e_pad = NW * CHUNK * c_chunks
    fill = jnp.full((e_pad - e_all,), n, edge_index.dtype)
    src_r = jnp.concatenate([edge_index[0], loop, fill]).reshape(NW, c_chunks, CHUNK)
    dst_r = jnp.concatenate([edge_index[1], loop, fill]).reshape(NW, c_chunks, CHUNK)
    cidx = jnp.stack([src_r, dst_r], axis=2)
    cidx = jnp.concatenate(
        [cidx, jnp.full((NW, 2, 2, CHUNK), n, edge_index.dtype)], axis=1)

    rb = lambda i: (i, 0)
    w_spec = pl.BlockSpec((D, D), lambda i: (0, 0))
    b_spec = pl.BlockSpec((1, D), lambda i: (0, 0))
    row_spec = pl.BlockSpec((ROW_BLK, D), rb)
    acc_spec = pl.BlockSpec((NC, ROW_BLK, D), lambda i: (0, i, 0))
    s_spec = pl.BlockSpec((NW, ROW_BLK), lambda i: (0, i))

    prep = _row_kernel(_prep_body, npad, 3,
                       [row_spec, w_spec, b_spec, w_spec, w_spec])
    h0, hs, hd = prep(xp, p["W_proj"], p["b_proj"].reshape(1, D),
                      p["gat_W_src"][0], p["gat_W_dst"][0])

    edge_k = _make_edge_kernel(c_chunks, npad)
    nl = p["gat_W_src"].shape[0]

    fin_mid = pl.pallas_call(
        _fin_mid_body,
        grid=(npad // ROW_BLK,),
        in_specs=[acc_spec, s_spec, b_spec, b_spec, b_spec, w_spec, w_spec],
        out_specs=[row_spec] * 3,
        out_shape=[jax.ShapeDtypeStruct((npad, D), jnp.float32)] * 3,
    )
    for l in range(nl - 1):
        acc, svec = edge_k(hs, hd, cidx, p["gat_att"][l])
        h, hs, hd = fin_mid(acc, svec, p["gat_bias"][l].reshape(1, D),
                            p["gat_ln_g"][l].reshape(1, D),
                            p["gat_ln_b"][l].reshape(1, D),
                            p["gat_W_src"][l + 1], p["gat_W_dst"][l + 1])

    acc, svec = edge_k(hs, hd, cidx, p["gat_att"][nl - 1])
    fin_last = pl.pallas_call(
        _fin_last_body,
        grid=(npad // ROW_BLK,),
        in_specs=[acc_spec, s_spec, b_spec, row_spec, w_spec, b_spec, b_spec,
                  b_spec],
        out_specs=[row_spec],
        out_shape=[jax.ShapeDtypeStruct((npad, D), jnp.float32)],
    )
    (xn,) = fin_last(acc, svec, p["gat_bias"][nl - 1].reshape(1, D), h0,
                     p["W_lin"], p["b_lin"].reshape(1, D),
                     p["bn_g"].reshape(1, D), p["bn_b"].reshape(1, D))

    X = xn[:n].reshape(b_graphs, seq_len, D)
    tok = lambda t: jnp.tile(t[None, None, :], (b_graphs, 1, 1))
    seq = jnp.concatenate([tok(p["CLS"]), X, tok(p["RING"]), tok(p["END"])],
                          axis=1)
    s_real = seq.shape[1]
    s_pad = 128
    seqp = jnp.zeros((b_graphs, s_pad, D), jnp.float32).at[:, :s_real].set(seq)
    mask = jnp.where(jnp.arange(s_pad) < s_real, 0.0, -1e30)
    mask = mask.astype(jnp.float32).reshape(1, s_pad)

    sblk = pl.BlockSpec((1, s_pad, D), lambda i: (i, 0, 0))
    full2 = lambda i: (0, 0)
    wspec = pl.BlockSpec((D, D), full2)
    bspec = pl.BlockSpec((1, D), full2)
    mspec = pl.BlockSpec((1, s_pad), full2)
    w1spec = pl.BlockSpec((D, 1024), full2)
    b1spec = pl.BlockSpec((1, 1024), full2)
    w2spec = pl.BlockSpec((1024, D), full2)
    tr = pl.pallas_call(
        _tr_body,
        grid=(b_graphs,),
        in_specs=[sblk, mspec,
                  wspec, bspec, wspec, bspec, wspec, bspec, wspec, bspec,
                  w1spec, b1spec, w2spec, bspec,
                  bspec, bspec, bspec, bspec],
        out_specs=[sblk],
        out_shape=[jax.ShapeDtypeStruct((b_graphs, s_pad, D), jnp.float32)],
    )
    (out,) = tr(seqp, mask,
                p["Wq"], p["bq"].reshape(1, D), p["Wk"], p["bk"].reshape(1, D),
                p["Wv"], p["bv"].reshape(1, D), p["Wo"], p["bo"].reshape(1, D),
                p["W1"], p["b1"].reshape(1, 1024), p["W2"], p["b2"].reshape(1, D),
                p["ln1_g"].reshape(1, D), p["ln1_b"].reshape(1, D),
                p["ln2_g"].reshape(1, D), p["ln2_b"].reshape(1, D))
    return out[:, :s_real]


# speedup vs baseline: 10.4818x; 1.0018x over previous
import functools
import math

import jax
import jax.numpy as jnp
from jax import lax
from jax.experimental import pallas as pl
from jax.experimental.pallas import tpu as pltpu
from jax.experimental.pallas import tpu_sc as plsc

D = 128
NC, NS = 2, 16
NW = NC * NS
CHUNK = 32
SUB = 16
NSUB = CHUNK // SUB
ROW_BLK = 1280


def _edge_body(C, hs_hbm, hd_hbm, cidx_hbm, att_hbm,
               feat_hbm, s_hbm,
               acc_sh, ibuf, ibuf2, dbuf, hs_buf, hd_buf, out_stage, att_v,
               zrow, t_ref, w_ref, s_loc, sg0h, sg0d, sg1h, sg1d, ssc, sidx):
    npad = hs_hbm.shape[0]
    cid = lax.axis_index("c")
    sid = lax.axis_index("s")
    wid = sid * NC + cid

    pltpu.sync_copy(att_hbm, att_v)

    z16 = jnp.zeros((16,), jnp.float32)
    for r in range(SUB):
        for c in range(D // 16):
            zrow[r, pl.ds(c * 16, 16)] = z16

    rows_per_sub = npad // NS

    def _zero(k, carry):
        pltpu.sync_copy(zrow, acc_sh.at[pl.ds(sid * rows_per_sub + k * SUB, SUB)])
        return carry
    lax.fori_loop(0, rows_per_sub // SUB, _zero, 0)

    def _zero_s(k, carry):
        s_loc[pl.ds(k * 16, 16)] = z16
        return carry
    lax.fori_loop(0, npad // 16, _zero_s, 0)

    z16i = jnp.zeros((16,), jnp.int32)
    for k in range(2 * CHUNK // 16):
        dbuf[pl.ds(k * 16, 16)] = z16i

    def _zero_o(r2, carry):
        for c in range(D // 16):
            out_stage[r2, pl.ds(c * 16, 16)] = z16
        return carry
    lax.fori_loop(0, 2 * CHUNK, _zero_o, 0)

    plsc.subcore_barrier()

    lanes = lax.iota(jnp.int32, SUB)
    g_h = (sg0h, sg1h)
    g_d = (sg0d, sg1d)

    pltpu.async_copy(out_stage, acc_sh.at[dbuf], ssc, add=True)

    pltpu.sync_copy(cidx_hbm.at[wid, pl.ds(0, 2)], ibuf)
    pltpu.sync_copy(hs_hbm.at[ibuf.at[0, 0]], hs_buf.at[0])
    pltpu.sync_copy(hd_hbm.at[ibuf.at[0, 1]], hd_buf.at[0])

    def _pair(i, ib_cur, ib_next):
        def _issue(ib, b, into):
            return (pltpu.async_copy(hs_hbm.at[ib.at[b, 0]], hs_buf.at[into],
                                     g_h[into]),
                    pltpu.async_copy(hd_hbm.at[ib.at[b, 1]], hd_buf.at[into],
                                     g_d[into]))

        d_b = _issue(ib_cur, 1, 1)

        pltpu.make_async_copy(out_stage, acc_sh.at[dbuf], ssc).wait()

        for b2 in range(2):
            for k in range(CHUNK // 16):
                dbuf[pl.ds(b2 * CHUNK + k * 16, 16)] = (
                    ib_cur[b2, 1, pl.ds(k * 16, 16)])

        d_i = pltpu.async_copy(cidx_hbm.at[wid, pl.ds(2 * i + 2, 2)],
                               ib_next, sidx)

        def _chunk_compute(b):
            def _sub(sb, cc):
                base = sb * SUB
                att_regs = [att_v[pl.ds(c * 16, 16)] for c in range(D // 16)]
                for e_i in range(SUB):
                    r = base + e_i
                    acc0 = None
                    acc1 = None
                    for c in range(D // 16):
                        a = hs_buf[b, r, pl.ds(c * 16, 16)]
                        t = a + hd_buf[b, r, pl.ds(c * 16, 16)]
                        zl = jnp.maximum(t, 0.2 * t)
                        term = zl * att_regs[c]
                        if c % 2 == 0:
                            acc0 = term if acc0 is None else acc0 + term
                        else:
                            acc1 = term if acc1 is None else acc1 + term
                    t_ref[e_i, :] = acc0 + acc1
                cols = [plsc.load_gather(t_ref,
                                         [lanes, jnp.full((SUB,), c, jnp.int32)])
                        for c in range(SUB)]
                while len(cols) > 1:
                    cols = [cols[j] + cols[j + 1] for j in range(0, len(cols), 2)]
                w_vec = jnp.exp(cols[0])
                w_ref[...] = w_vec
                dst_vec = dbuf[pl.ds(b * CHUNK + base, SUB)]
                for li in range(SUB):
                    plsc.addupdate_scatter(s_loc, [dst_vec], w_vec,
                                           mask=lanes == li)
                for e_i in range(SUB):
                    ws = plsc.load_gather(
                        w_ref, [jnp.full((SUB,), e_i, jnp.int32)])
                    r = base + e_i
                    ro = b * CHUNK + base + e_i
                    for c in range(D // 16):
                        out_stage[ro, pl.ds(c * 16, 16)] = (
                            hs_buf[b, r, pl.ds(c * 16, 16)] * ws)
                return cc
            lax.fori_loop(0, NSUB, _sub, 0)

        _chunk_compute(0)
        d_b[0].wait()
        d_b[1].wait()
        d_i.wait()
        d_a2 = _issue(ib_next, 0, 0)
        _chunk_compute(1)
        pltpu.async_copy(out_stage, acc_sh.at[dbuf], ssc, add=True)
        d_a2[0].wait()
        d_a2[1].wait()

    def _loop(j, carry):
        _pair(2 * j, ibuf, ibuf2)
        _pair(2 * j + 1, ibuf2, ibuf)
        return carry
    lax.fori_loop(0, C // 4, _loop, 0)

    pltpu.make_async_copy(out_stage, acc_sh.at[dbuf], ssc).wait()
    plsc.subcore_barrier()
    pltpu.sync_copy(acc_sh.at[pl.ds(sid * rows_per_sub, rows_per_sub)],
                    feat_hbm.at[cid, pl.ds(sid * rows_per_sub, rows_per_sub)])
    pltpu.sync_copy(s_loc, s_hbm.at[wid])


@functools.cache
def _make_edge_kernel(C, npad):
    mesh = plsc.VectorSubcoreMesh(core_axis_name="c", subcore_axis_name="s")
    return pl.kernel(
        functools.partial(_edge_body, C),
        out_type=(jax.ShapeDtypeStruct((NC, npad, D), jnp.float32),
                  jax.ShapeDtypeStruct((NW, npad), jnp.float32)),
        mesh=mesh,
        compiler_params=pltpu.CompilerParams(needs_layout_passes=False),
        scratch_types=[
            pltpu.VMEM_SHARED((npad, D), jnp.float32),
            pltpu.VMEM((2, 2, CHUNK), jnp.int32),
            pltpu.VMEM((2, 2, CHUNK), jnp.int32),
            pltpu.VMEM((2 * CHUNK,), jnp.int32),
            pltpu.VMEM((2, CHUNK, D), jnp.float32),
            pltpu.VMEM((2, CHUNK, D), jnp.float32),
            pltpu.VMEM((2 * CHUNK, D), jnp.float32),
            pltpu.VMEM((D,), jnp.float32),
            pltpu.VMEM((SUB, D), jnp.float32),
            pltpu.VMEM((SUB, SUB), jnp.float32),
            pltpu.VMEM((SUB,), jnp.float32),
            pltpu.VMEM((npad,), jnp.float32),
            pltpu.SemaphoreType.DMA,
            pltpu.SemaphoreType.DMA,
            pltpu.SemaphoreType.DMA,
            pltpu.SemaphoreType.DMA,
            pltpu.SemaphoreType.DMA,
            pltpu.SemaphoreType.DMA,
        ],
    )


def _prep_body(x_ref, wp_ref, bp_ref, ws_ref, wd_ref, h0_ref, hs_ref, hd_ref):
    h0 = jnp.dot(x_ref[...], wp_ref[...],
                 preferred_element_type=jnp.float32) + bp_ref[...]
    h0_ref[...] = h0
    hs_ref[...] = jnp.dot(h0, ws_ref[...], preferred_element_type=jnp.float32)
    hd_ref[...] = jnp.dot(h0, wd_ref[...], preferred_element_type=jnp.float32)


def _ln_rows(x, g, b):
    m = jnp.mean(x, axis=-1, keepdims=True)
    v = jnp.mean((x - m) ** 2, axis=-1, keepdims=True)
    return (x - m) / jnp.sqrt(v + 1e-5) * g + b


def _fin_mid_body(acc_ref, s_ref, bias_ref, g_ref, b_ref, ws_ref, wd_ref,
                  h_ref, hs_ref, hd_ref):
    a = acc_ref[0] + acc_ref[1]
    s = jnp.sum(s_ref[...], axis=0)[:, None]
    out = a / (s + 1e-30) + bias_ref[...]
    h = jnp.maximum(_ln_rows(out, g_ref[...], b_ref[...]), 0.0)
    h_ref[...] = h
    hs_ref[...] = jnp.dot(h, ws_ref[...], preferred_element_type=jnp.float32)
    hd_ref[...] = jnp.dot(h, wd_ref[...], preferred_element_type=jnp.float32)


def _fin_last_body(acc_ref, s_ref, bias_ref, h0_ref, wl_ref, bl_ref, bng_ref,
                   bnb_ref, xn_ref):
    a = acc_ref[0] + acc_ref[1]
    s = jnp.sum(s_ref[...], axis=0)[:, None]
    h = a / (s + 1e-30) + bias_ref[...]
    xn = jnp.dot(h0_ref[...] + h, wl_ref[...],
                 preferred_element_type=jnp.float32) + bl_ref[...]
    xn_ref[...] = xn / jnp.sqrt(1.0 + 1e-5) * bng_ref[...] + bnb_ref[...]


def _tr_body(seq_ref, mask_ref, wq_ref, bq_ref, wk_ref, bk_ref, wv_ref, bv_ref,
             wo_ref, bo_ref, w1_ref, b1_ref, w2_ref, b2_ref,
             g1_ref, gb1_ref, g2_ref, gb2_ref, out_ref):
    xx = seq_ref[0]
    q = jnp.dot(xx, wq_ref[...], preferred_element_type=jnp.float32) + bq_ref[...]
    k = jnp.dot(xx, wk_ref[...], preferred_element_type=jnp.float32) + bk_ref[...]
    v = jnp.dot(xx, wv_ref[...], preferred_element_type=jnp.float32) + bv_ref[...]
    dh = 32
    scale = 1.0 / math.sqrt(float(dh))
    outs = []
    for h in range(4):
        qh = q[:, h * dh:(h + 1) * dh]
        kh = k[:, h * dh:(h + 1) * dh]
        vh = v[:, h * dh:(h + 1) * dh]
        sc = lax.dot_general(qh, kh, (((1,), (1,)), ((), ())),
                             preferred_element_type=jnp.float32) * scale
        sc = sc + mask_ref[...]
        m = jnp.max(sc, axis=1, keepdims=True)
        p = jnp.exp(sc - m)
        p = p / jnp.sum(p, axis=1, keepdims=True)
        outs.append(jnp.dot(p, vh, preferred_element_type=jnp.float32))
    sa = jnp.concatenate(outs, axis=1)
    sa = jnp.dot(sa, wo_ref[...], preferred_element_type=jnp.float32) + bo_ref[...]
    s1 = _ln_rows(xx + sa, g1_ref[...], gb1_ref[...])
    ff = jnp.maximum(jnp.dot(s1, w1_ref[...],
                             preferred_element_type=jnp.float32) + b1_ref[...], 0.0)
    ff = jnp.dot(ff, w2_ref[...], preferred_element_type=jnp.float32) + b2_ref[...]
    out_ref[0] = _ln_rows(s1 + ff, g2_ref[...], gb2_ref[...])


def _row_kernel(body, npad, n_out, extra_specs, grid_rows=ROW_BLK):
    ngrid = npad // grid_rows
    rb = lambda i: (i, 0)
    full2 = lambda i: (0, 0)
    return pl.pallas_call(
        body,
        grid=(ngrid,),
        in_specs=extra_specs,
        out_specs=[pl.BlockSpec((grid_rows, D), rb)] * n_out,
        out_shape=[jax.ShapeDtypeStruct((npad, D), jnp.float32)] * n_out,
    )


def kernel(x, edge_index, ptr, params):
    p = params
    n = x.shape[0]
    npad = ((n + NS * SUB * 40 - 1) // (NS * SUB * 40)) * (NS * SUB * 40)
    b_graphs = ptr.shape[0] - 1
    seq_len = n // b_graphs

    xp = jnp.zeros((npad, D), jnp.float32).at[:n].set(x)
    loop = jnp.arange(n, dtype=edge_index.dtype)
    e_all = edge_index.shape[1] + n
    c_chunks = -(-e_all // (NW * CHUNK))
    e_pad = NW * CHUNK * c_chunks
    fill = jnp.full((e_pad - e_all,), n, edge_index.dtype)
    src_r = jnp.concatenate([edge_index[0], loop, fill]).reshape(NW, c_chunks, CHUNK)
    dst_r = jnp.concatenate([edge_index[1], loop, fill]).reshape(NW, c_chunks, CHUNK)
    cidx = jnp.stack([src_r, dst_r], axis=2)
    cidx = jnp.concatenate(
        [cidx, jnp.full((NW, 2, 2, CHUNK), n, edge_index.dtype)], axis=1)

    rb = lambda i: (i, 0)
    w_spec = pl.BlockSpec((D, D), lambda i: (0, 0))
    b_spec = pl.BlockSpec((1, D), lambda i: (0, 0))
    row_spec = pl.BlockSpec((ROW_BLK, D), rb)
    acc_spec = pl.BlockSpec((NC, ROW_BLK, D), lambda i: (0, i, 0))
    s_spec = pl.BlockSpec((NW, ROW_BLK), lambda i: (0, i))

    prep = _row_kernel(_prep_body, npad, 3,
                       [row_spec, w_spec, b_spec, w_spec, w_spec])
    h0, hs, hd = prep(xp, p["W_proj"], p["b_proj"].reshape(1, D),
                      p["gat_W_src"][0], p["gat_W_dst"][0])

    edge_k = _make_edge_kernel(c_chunks, npad)
    nl = p["gat_W_src"].shape[0]

    fin_mid = pl.pallas_call(
        _fin_mid_body,
        grid=(npad // ROW_BLK,),
        in_specs=[acc_spec, s_spec, b_spec, b_spec, b_spec, w_spec, w_spec],
        out_specs=[row_spec] * 3,
        out_shape=[jax.ShapeDtypeStruct((npad, D), jnp.float32)] * 3,
    )
    for l in range(nl - 1):
        acc, svec = edge_k(hs, hd, cidx, p["gat_att"][l])
        h, hs, hd = fin_mid(acc, svec, p["gat_bias"][l].reshape(1, D),
                            p["gat_ln_g"][l].reshape(1, D),
                            p["gat_ln_b"][l].reshape(1, D),
                            p["gat_W_src"][l + 1], p["gat_W_dst"][l + 1])

    acc, svec = edge_k(hs, hd, cidx, p["gat_att"][nl - 1])
    fin_last = pl.pallas_call(
        _fin_last_body,
        grid=(npad // ROW_BLK,),
        in_specs=[acc_spec, s_spec, b_spec, row_spec, w_spec, b_spec, b_spec,
                  b_spec],
        out_specs=[row_spec],
        out_shape=[jax.ShapeDtypeStruct((npad, D), jnp.float32)],
    )
    (xn,) = fin_last(acc, svec, p["gat_bias"][nl - 1].reshape(1, D), h0,
                     p["W_lin"], p["b_lin"].reshape(1, D),
                     p["bn_g"].reshape(1, D), p["bn_b"].reshape(1, D))

    X = xn[:n].reshape(b_graphs, seq_len, D)
    tok = lambda t: jnp.tile(t[None, None, :], (b_graphs, 1, 1))
    seq = jnp.concatenate([tok(p["CLS"]), X, tok(p["RING"]), tok(p["END"])],
                          axis=1)
    s_real = seq.shape[1]
    s_pad = 128
    seqp = jnp.zeros((b_graphs, s_pad, D), jnp.float32).at[:, :s_real].set(seq)
    mask = jnp.where(jnp.arange(s_pad) < s_real, 0.0, -1e30)
    mask = mask.astype(jnp.float32).reshape(1, s_pad)

    sblk = pl.BlockSpec((1, s_pad, D), lambda i: (i, 0, 0))
    full2 = lambda i: (0, 0)
    wspec = pl.BlockSpec((D, D), full2)
    bspec = pl.BlockSpec((1, D), full2)
    mspec = pl.BlockSpec((1, s_pad), full2)
    w1spec = pl.BlockSpec((D, 1024), full2)
    b1spec = pl.BlockSpec((1, 1024), full2)
    w2spec = pl.BlockSpec((1024, D), full2)
    tr = pl.pallas_call(
        _tr_body,
        grid=(b_graphs,),
        in_specs=[sblk, mspec,
                  wspec, bspec, wspec, bspec, wspec, bspec, wspec, bspec,
                  w1spec, b1spec, w2spec, bspec,
                  bspec, bspec, bspec, bspec],
        out_specs=[sblk],
        out_shape=[jax.ShapeDtypeStruct((b_graphs, s_pad, D), jnp.float32)],
    )
    (out,) = tr(seqp, mask,
                p["Wq"], p["bq"].reshape(1, D), p["Wk"], p["bk"].reshape(1, D),
                p["Wv"], p["bv"].reshape(1, D), p["Wo"], p["bo"].reshape(1, D),
                p["W1"], p["b1"].reshape(1, 1024), p["W2"], p["b2"].reshape(1, D),
                p["ln1_g"].reshape(1, D), p["ln1_b"].reshape(1, D),
                p["ln2_g"].reshape(1, D), p["ln2_b"].reshape(1, D))
    return out[:, :s_real]
